# Initial kernel scaffold; baseline (speedup 1.0000x reference)
import jax
import jax.numpy as jnp
from jax.experimental import pallas as pl


def kernel(x, edge_index, W_cheb, b_cheb, gn_weight, gn_bias, gn_mean_scale, out_W, out_b):
    raise NotImplementedError("write your pallas kernel here")


# speedup vs baseline: 5.0424x; 5.0424x over previous
import functools

import jax
import jax.numpy as jnp
from jax import lax
from jax.experimental import pallas as pl
from jax.experimental.pallas import tpu as pltpu
from jax.experimental.pallas import tpu_sc as plsc

_N = 10000
_E = 320000
_D = 128
_B = 16
_EPS = 1e-5

_NC = 2
_NS = 16
_HALF = _N // 2

_EPT = _E // _NS
_CH = 32
_IB = 800
_NIB = _EPT // _IB
_CPB = _IB // _CH
_ACC_R = 5024
_ZCH = _ACC_R // _CH
_OCH = 40
_NOCH = _HALF // _OCH

_NW = _NC * _NS
_EPW = _E // _NW
_DCH = 80
_DNCH = _EPW // _DCH
_NZCH = _N // _DCH

_sc_mesh = plsc.VectorSubcoreMesh(core_axis_name="c", subcore_axis_name="s")

_f32 = jnp.float32


def _zero_vmem_2d(ref, nrows):
    zer = jnp.zeros((16,), _f32)

    def body(r, _):
        for cc in range(_D // 16):
            ref[r, pl.ds(cc * 16, 16)] = zer
        return 0

    lax.fori_loop(0, nrows, body, 0)


def _fill_vmem_1d(ref, n, value):
    for i in range(n // 16):
        ref[pl.ds(i * 16, 16)] = jnp.full((16,), value, _f32)


@functools.partial(
    pl.kernel,
    out_type=(
        jax.ShapeDtypeStruct((_N,), _f32),
        jax.ShapeDtypeStruct((_N,), _f32),
    ),
    mesh=_sc_mesh,
    scratch_types=[
        pltpu.VMEM((_DNCH, _DCH), jnp.int32),
        pltpu.VMEM((_DCH,), _f32),
        pltpu.VMEM((_DCH,), _f32),
        pltpu.VMEM_SHARED((_N,), _f32),
    ],
)
def _sc_deg(src_hbm, out0, out1, srcv, onesv, zbuf, acc):
    c = lax.axis_index("c")
    s = lax.axis_index("s")
    wid = c * _NS + s
    pltpu.sync_copy(src_hbm.at[wid], srcv)
    _fill_vmem_1d(onesv, _DCH, 1.0)
    _fill_vmem_1d(zbuf, _DCH, 0.0)

    @pl.when(s == 0)
    def _():
        def zb(j, _):
            pltpu.sync_copy(zbuf, acc.at[pl.ds(j * _DCH, _DCH)])
            return 0
        lax.fori_loop(0, _NZCH, zb, 0)

    plsc.subcore_barrier()

    def body(j, _):
        pltpu.sync_copy(onesv, acc.at[srcv.at[j]], add=True)
        return 0

    lax.fori_loop(0, _DNCH, body, 0)
    plsc.subcore_barrier()

    @pl.when(s == 0)
    def _():
        def cp(j, _):
            pltpu.sync_copy(acc.at[pl.ds(j * _DCH, _DCH)], zbuf)

            @pl.when(c == 0)
            def _():
                pltpu.sync_copy(zbuf, out0.at[pl.ds(j * _DCH, _DCH)])

            @pl.when(c == 1)
            def _():
                pltpu.sync_copy(zbuf, out1.at[pl.ds(j * _DCH, _DCH)])

            return 0
        lax.fori_loop(0, _NZCH, cp, 0)


@functools.partial(
    pl.kernel,
    out_type=(
        jax.ShapeDtypeStruct((_HALF, _D), _f32),
        jax.ShapeDtypeStruct((_HALF, _D), _f32),
    ),
    mesh=_sc_mesh,
    scratch_types=[
        pltpu.VMEM((_IB,), jnp.int32),
        pltpu.VMEM((_IB,), jnp.int32),
        pltpu.VMEM((_CH, _D), _f32),
        pltpu.VMEM((_CH, _D), _f32),
        pltpu.VMEM_SHARED((_ACC_R, _D), _f32),
        pltpu.SemaphoreType.DMA,
        pltpu.SemaphoreType.DMA,
    ],
)
def _sc_scatter(gst_hbm, gi0_hbm, gi1_hbm, df_hbm, out0, out1, srcv, dstv,
                rows0, rows1, acc, sem0, sem1):
    c = lax.axis_index("c")
    s = lax.axis_index("s")

    _zero_vmem_2d(rows0, _CH)

    def zchunk(k, _):
        cid = s + _NS * k

        @pl.when(cid < _ZCH)
        def _():
            pltpu.sync_copy(rows0, acc.at[pl.ds(cid * _CH, _CH)])

        return 0

    lax.fori_loop(0, _ZCH // _NS + 1, zchunk, 0)
    plsc.subcore_barrier()

    def block(b, _):
        off = s * _EPT + b * _IB

        @pl.when(c == 0)
        def _():
            pltpu.sync_copy(gi0_hbm.at[pl.ds(off, _IB)], srcv)

        @pl.when(c == 1)
        def _():
            pltpu.sync_copy(gi1_hbm.at[pl.ds(off, _IB)], srcv)

        pltpu.sync_copy(df_hbm.at[pl.ds(off, _IB)], dstv)

        def chunk(k, _):
            gidx = srcv.at[pl.ds(k * _CH, _CH)]
            pltpu.async_copy(gst_hbm.at[gidx], rows0, sem0).wait()
            for q in range(_CH // 16):
                idx16 = dstv[pl.ds(k * _CH + q * 16, 16)]
                pltpu.sync_copy(rows0.at[pl.ds(q * 16, 16)],
                                acc.at[idx16], add=True)
            return 0

        lax.fori_loop(0, _CPB, chunk, 0)
        return 0

    lax.fori_loop(0, _NIB, block, 0)
    plsc.subcore_barrier()

    def ochunk(k, _):
        cid = s + _NS * k

        @pl.when(cid < _NOCH)
        def _():
            sl = pl.ds(cid * _OCH, _OCH)

            @pl.when(c == 0)
            def _():
                pltpu.sync_copy(acc.at[sl], out0.at[sl])

            @pl.when(c == 1)
            def _():
                pltpu.sync_copy(acc.at[sl], out1.at[sl])

        return 0

    lax.fori_loop(0, _NOCH // _NS + 1, ochunk, 0)


_GRID = 10
_RBLK = _N // _GRID
_IBLK = _E // _D // _GRID


def _dot(a, b, dims):
    return lax.dot_general(a, b, (dims, ((), ())),
                           precision=lax.Precision.HIGHEST,
                           preferred_element_type=_f32)


def _half_map(i):
    return (lax.rem(i, _GRID // 2), 0)


def _tc_prep_body(dega_ref, degb_ref, feats_ref, src_ref, dst_ref,
                  dis_ref, gst_ref, gi1_ref, df_ref):
    deg = dega_ref[...] + degb_ref[...]
    safe = jnp.where(deg > 0, deg, 1.0)
    dis = jnp.where(deg > 0, lax.rsqrt(safe), 0.0)
    dis_ref[...] = dis
    g = feats_ref[...] * dis
    gst_ref[0] = g
    gst_ref[1] = -g

    @pl.when(pl.program_id(0) == 0)
    def _():
        srci = src_ref[...]
        dsti = dst_ref[...]
        hi = dsti >= _HALF
        gi1_ref[...] = jnp.where(hi, srci + _N, srci)
        df_ref[...] = jnp.where(hi, dsti - _HALF, dsti)


_tc_prep = pl.pallas_call(
    _tc_prep_body,
    grid=(_GRID,),
    in_specs=[
        pl.BlockSpec((_RBLK, 1), lambda i: (i, 0)),
        pl.BlockSpec((_RBLK, 1), lambda i: (i, 0)),
        pl.BlockSpec((_RBLK, _D), lambda i: (i, 0)),
        pl.BlockSpec((_E // _D, _D), lambda i: (0, 0)),
        pl.BlockSpec((_E // _D, _D), lambda i: (0, 0)),
    ],
    out_specs=[
        pl.BlockSpec((_RBLK, 1), lambda i: (i, 0)),
        pl.BlockSpec((2, _RBLK, _D), lambda i: (0, i, 0)),
        pl.BlockSpec((_E // _D, _D), lambda i: (0, 0)),
        pl.BlockSpec((_E // _D, _D), lambda i: (0, 0)),
    ],
    out_shape=[
        jax.ShapeDtypeStruct((_N, 1), _f32),
        jax.ShapeDtypeStruct((2, _N, _D), _f32),
        jax.ShapeDtypeStruct((_E // _D, _D), jnp.int32),
        jax.ShapeDtypeStruct((_E // _D, _D), jnp.int32),
    ],
)


def _fold_decode(a0, a1, i):
    sign = jnp.where(i < _GRID // 2, 0.5, -0.5)
    return 0.5 * a0 + sign * a1


def _tc_mid_body(a0_ref, a1_ref, dis_ref, feats_ref, w0_ref, w1_ref,
                 out01_ref, gst1_ref):
    i = pl.program_id(0)
    s0 = _fold_decode(a0_ref[...], a1_ref[...], i)
    dis = dis_ref[...]
    tx1 = -dis * s0
    out01_ref[...] = (_dot(feats_ref[...], w0_ref[...], ((1,), (0,)))
                      + _dot(tx1, w1_ref[...], ((1,), (0,))))
    g1 = dis * tx1
    gst1_ref[0] = g1
    gst1_ref[1] = -g1


_tc_mid = pl.pallas_call(
    _tc_mid_body,
    grid=(_GRID,),
    in_specs=[
        pl.BlockSpec((_RBLK, _D), _half_map),
        pl.BlockSpec((_RBLK, _D), _half_map),
        pl.BlockSpec((_RBLK, 1), lambda i: (i, 0)),
        pl.BlockSpec((_RBLK, _D), lambda i: (i, 0)),
        pl.BlockSpec((_D, _D), lambda i: (0, 0)),
        pl.BlockSpec((_D, _D), lambda i: (0, 0)),
    ],
    out_specs=[
        pl.BlockSpec((_RBLK, _D), lambda i: (i, 0)),
        pl.BlockSpec((2, _RBLK, _D), lambda i: (0, i, 0)),
    ],
    out_shape=[
        jax.ShapeDtypeStruct((_N, _D), _f32),
        jax.ShapeDtypeStruct((2, _N, _D), _f32),
    ],
)


def _tc_final_body(a0_ref, a1_ref, dis_ref, feats_ref, out01_ref, batch_ref,
                   w2_ref, bch_ref, gnw_ref, gnb_ref, gnms_ref, ow_ref, ob_ref,
                   out_ref, sums1, sums2, cnt, maxz):
    i = pl.program_id(0)

    @pl.when(i == 0)
    def _():
        sums1[...] = jnp.zeros_like(sums1)
        sums2[...] = jnp.zeros_like(sums2)
        cnt[...] = jnp.zeros_like(cnt)
        maxz[...] = jnp.full_like(maxz, -3.0e38)

    s1 = _fold_decode(a0_ref[...], a1_ref[...], i)
    tx2 = -2.0 * dis_ref[...] * s1 - feats_ref[...]
    x1 = out01_ref[...] + _dot(tx2, w2_ref[...], ((1,), (0,))) + bch_ref[...]
    x1 = jnp.clip(x1, -1.0, 1.0)

    bcol = batch_ref[...]
    iota = lax.broadcasted_iota(jnp.int32, (1, _B), 1).astype(_f32)
    onehot = (bcol == iota).astype(_f32)
    cnt[...] += _dot(onehot, jnp.ones((_RBLK, 1), _f32), ((0,), (0,)))
    sums1[...] += _dot(onehot, x1, ((0,), (0,)))
    sums2[...] += _dot(onehot, x1 * x1, ((0,), (0,)))

    sign = jnp.where(gnw_ref[...] >= 0, 1.0, -1.0)
    z = x1 * sign
    for b in range(_B):
        mb = jnp.max(jnp.where(bcol == float(b), z, -3.0e38), axis=0)
        maxz[b, :] = jnp.maximum(maxz[b, :], mb)

    @pl.when(i == _GRID - 1)
    def _():
        cn = cnt[...]
        m1 = sums1[...] / cn
        m2 = sums2[...] / cn
        ms = gnms_ref[...]
        w = gnw_ref[...]
        var = m2 - (2.0 * ms - ms * ms) * m1 * m1
        std = jnp.sqrt(var + _EPS)
        pooled = (jnp.abs(w) * maxz[...] - w * ms * m1) / std + gnb_ref[...]
        out_ref[...] = _dot(pooled, ow_ref[...], ((1,), (1,))) + ob_ref[...]


_tc_final = pl.pallas_call(
    _tc_final_body,
    grid=(_GRID,),
    in_specs=[
        pl.BlockSpec((_RBLK, _D), _half_map),
        pl.BlockSpec((_RBLK, _D), _half_map),
        pl.BlockSpec((_RBLK, 1), lambda i: (i, 0)),
        pl.BlockSpec((_RBLK, _D), lambda i: (i, 0)),
        pl.BlockSpec((_RBLK, _D), lambda i: (i, 0)),
        pl.BlockSpec((_RBLK, 1), lambda i: (i, 0)),
        pl.BlockSpec((_D, _D), lambda i: (0, 0)),
        pl.BlockSpec((1, _D), lambda i: (0, 0)),
        pl.BlockSpec((1, _D), lambda i: (0, 0)),
        pl.BlockSpec((1, _D), lambda i: (0, 0)),
        pl.BlockSpec((1, _D), lambda i: (0, 0)),
        pl.BlockSpec((_D, _D), lambda i: (0, 0)),
        pl.BlockSpec((1, _D), lambda i: (0, 0)),
    ],
    out_specs=pl.BlockSpec((_B, _D), lambda i: (0, 0)),
    out_shape=jax.ShapeDtypeStruct((_B, _D), _f32),
    scratch_shapes=[
        pltpu.VMEM((_B, _D), _f32),
        pltpu.VMEM((_B, _D), _f32),
        pltpu.VMEM((_B, 1), _f32),
        pltpu.VMEM((_B, _D), _f32),
    ],
)


def kernel(x, edge_index, W_cheb, b_cheb, gn_weight, gn_bias, gn_mean_scale,
           out_W, out_b):
    feats = x[:, :_D]
    batchf = x[:, -1:]
    src = edge_index[0]
    dst = edge_index[1]
    srcd = src.reshape(_NW, _DNCH, _DCH)
    src2 = src.reshape(_E // _D, _D)
    dst2 = dst.reshape(_E // _D, _D)
    gi0 = src

    dega, degb = _sc_deg(srcd)
    dis, gst0, gi1, df = _tc_prep(dega.reshape(_N, 1), degb.reshape(_N, 1),
                                  feats, src2, dst2)
    gi1 = gi1.reshape(_E)
    df = df.reshape(_E)

    a0, a1 = _sc_scatter(gst0.reshape(2 * _N, _D), gi0, gi1, df)
    out01, gst1 = _tc_mid(a0, a1, dis, feats, W_cheb[0], W_cheb[1])
    b0, b1 = _sc_scatter(gst1.reshape(2 * _N, _D), gi0, gi1, df)
    row = lambda v: v.reshape(1, _D)
    return _tc_final(b0, b1, dis, feats, out01, batchf, W_cheb[2],
                     row(b_cheb), row(gn_weight), row(gn_bias),
                     row(gn_mean_scale), out_W, row(out_b))


# speedup vs baseline: 9.5039x; 1.8848x over previous
import functools

import jax
import jax.numpy as jnp
from jax import lax
from jax.experimental import pallas as pl
from jax.experimental.pallas import tpu as pltpu
from jax.experimental.pallas import tpu_sc as plsc

_N = 10000
_E = 320000
_D = 128
_B = 16
_EPS = 1e-5

_NC = 2
_NS = 16
_HALF = _N // 2

_EPT = _E // _NS
_CH = 32
_NCH = _EPT // _CH
_IB = 800
_CPB = _IB // _CH
_NSLAB = _EPT // _IB
_ACC_R = 5008
_ZCH = _ACC_R // 16
_OCH = 40
_NOCH = _HALF // _OCH

_NW = _NC * _NS
_EPW = _E // _NW
_DCH = 80
_DNCH = _EPW // _DCH
_NZCH = _N // _DCH

_sc_mesh = plsc.VectorSubcoreMesh(core_axis_name="c", subcore_axis_name="s")

_f32 = jnp.float32


def _zero_vmem_2d(ref, nrows):
    zer = jnp.zeros((16,), _f32)

    def body(r, _):
        for cc in range(_D // 16):
            ref[r, pl.ds(cc * 16, 16)] = zer
        return 0

    lax.fori_loop(0, nrows, body, 0)


def _fill_vmem_1d(ref, n, value):
    for i in range(n // 16):
        ref[pl.ds(i * 16, 16)] = jnp.full((16,), value, _f32)


@functools.partial(
    pl.kernel,
    out_type=(
        jax.ShapeDtypeStruct((_N,), _f32),
        jax.ShapeDtypeStruct((_N,), _f32),
    ),
    mesh=_sc_mesh,
    scratch_types=[
        pltpu.VMEM((_DNCH, _DCH), jnp.int32),
        pltpu.VMEM((_DCH,), _f32),
        pltpu.VMEM((_DCH,), _f32),
        pltpu.VMEM_SHARED((_N,), _f32),
    ],
)
def _sc_deg(src_hbm, out0, out1, srcv, onesv, zbuf, acc):
    c = lax.axis_index("c")
    s = lax.axis_index("s")
    wid = c * _NS + s
    pltpu.sync_copy(src_hbm.at[wid], srcv)
    _fill_vmem_1d(onesv, _DCH, 1.0)
    _fill_vmem_1d(zbuf, _DCH, 0.0)

    @pl.when(s == 0)
    def _():
        def zb(j, _):
            pltpu.sync_copy(zbuf, acc.at[pl.ds(j * _DCH, _DCH)])
            return 0
        lax.fori_loop(0, _NZCH, zb, 0)

    plsc.subcore_barrier()

    def body(j, _):
        pltpu.sync_copy(onesv, acc.at[srcv.at[j]], add=True)
        return 0

    lax.fori_loop(0, _DNCH, body, 0)
    plsc.subcore_barrier()

    @pl.when(s == 0)
    def _():
        def cp(j, _):
            pltpu.sync_copy(acc.at[pl.ds(j * _DCH, _DCH)], zbuf)

            @pl.when(c == 0)
            def _():
                pltpu.sync_copy(zbuf, out0.at[pl.ds(j * _DCH, _DCH)])

            @pl.when(c == 1)
            def _():
                pltpu.sync_copy(zbuf, out1.at[pl.ds(j * _DCH, _DCH)])

            return 0
        lax.fori_loop(0, _NZCH, cp, 0)


@functools.partial(
    pl.kernel,
    out_type=(
        jax.ShapeDtypeStruct((_HALF, _D), _f32),
        jax.ShapeDtypeStruct((_HALF, _D), _f32),
    ),
    mesh=_sc_mesh,
    scratch_types=[
        pltpu.VMEM((2 * _IB,), jnp.int32),
        pltpu.VMEM((2 * _IB,), jnp.int32),
        pltpu.VMEM((2 * _CH, _D), _f32),
        pltpu.VMEM_SHARED((_ACC_R, _D), _f32),
        pltpu.SemaphoreType.DMA,
        pltpu.SemaphoreType.DMA,
        pltpu.SemaphoreType.DMA,
        pltpu.SemaphoreType.DMA,
        pltpu.SemaphoreType.DMA,
    ],
)
def _sc_scatter(gst_hbm, gi0_hbm, gi1_hbm, df_hbm, out0, out1, srcv, dstv,
                rows, acc, gsem0, gsem1, ssem0, ssem1, isem):
    c = lax.axis_index("c")
    s = lax.axis_index("s")

    _zero_vmem_2d(rows, 16)

    def zchunk(k, _):
        cid = s + _NS * k

        @pl.when(cid < _ZCH)
        def _():
            pltpu.sync_copy(rows.at[pl.ds(0, 16)], acc.at[pl.ds(cid * 16, 16)])

        return 0

    lax.fori_loop(0, _ZCH // _NS + 1, zchunk, 0)
    plsc.subcore_barrier()

    base = s * _EPT

    def load_slab(m, half, sem):
        off = base + m * _IB
        dst_sl = pl.ds(half * _IB, _IB)
        if sem is None:
            @pl.when(c == 0)
            def _():
                pltpu.sync_copy(gi0_hbm.at[pl.ds(off, _IB)], srcv.at[dst_sl])

            @pl.when(c == 1)
            def _():
                pltpu.sync_copy(gi1_hbm.at[pl.ds(off, _IB)], srcv.at[dst_sl])

            pltpu.sync_copy(df_hbm.at[pl.ds(off, _IB)], dstv.at[dst_sl])
        else:
            @pl.when(c == 0)
            def _():
                pltpu.async_copy(gi0_hbm.at[pl.ds(off, _IB)], srcv.at[dst_sl],
                                 sem)

            @pl.when(c == 1)
            def _():
                pltpu.async_copy(gi1_hbm.at[pl.ds(off, _IB)], srcv.at[dst_sl],
                                 sem)

            pltpu.async_copy(df_hbm.at[pl.ds(off, _IB)], dstv.at[dst_sl], isem)

    load_slab(0, 0, None)
    load_slab(1, 1, isem)
    pltpu.async_copy(gst_hbm.at[srcv.at[pl.ds(0, _CH)]],
                     rows.at[pl.ds(0, _CH)], gsem0)

    def _idx_off(k):
        m = k // _CPB
        return lax.rem(m, 2) * _IB + (k - m * _CPB) * _CH

    def chunk(k, _):
        m = k // _CPB
        pos = k - m * _CPB
        sb = _idx_off(k)
        sb1 = _idx_off(k + 1)

        @pl.when((pos == 0) & (k >= _CPB) & (k + _CPB < _NCH))
        def _():
            load_slab(m + 1, lax.rem(m + 1, 2), isem)

        @pl.when((pos == _CPB - 1) & (k + 1 < _NCH))
        def _():
            pltpu.make_async_copy(
                df_hbm.at[pl.ds(base, _IB)],
                srcv.at[pl.ds(lax.rem(m + 1, 2) * _IB, _IB)], isem).wait()
            pltpu.make_async_copy(
                df_hbm.at[pl.ds(base, _IB)],
                dstv.at[pl.ds(lax.rem(m + 1, 2) * _IB, _IB)], isem).wait()

        def work(roff, gsem, ssem, roff_n, gsem_n, ssem_n):
            @pl.when(k >= 1)
            def _():
                for q in range(2):
                    dummy = dstv[pl.ds(sb + q * 16, 16)]
                    pltpu.make_async_copy(
                        rows.at[pl.ds(roff_n + q * 16, 16)],
                        acc.at[dummy], ssem_n).wait()

            @pl.when(k + 1 < _NCH)
            def _():
                pltpu.async_copy(gst_hbm.at[srcv.at[pl.ds(sb1, _CH)]],
                                 rows.at[pl.ds(roff_n, _CH)], gsem_n)

            pltpu.make_async_copy(gst_hbm.at[srcv.at[pl.ds(sb, _CH)]],
                                  rows.at[pl.ds(roff, _CH)], gsem).wait()
            for q in range(2):
                idx16 = dstv[pl.ds(sb + q * 16, 16)]
                pltpu.async_copy(rows.at[pl.ds(roff + q * 16, 16)],
                                 acc.at[idx16], ssem, add=True)

        @pl.when(lax.rem(k, 2) == 0)
        def _():
            work(0, gsem0, ssem0, _CH, gsem1, ssem1)

        @pl.when(lax.rem(k, 2) == 1)
        def _():
            work(_CH, gsem1, ssem1, 0, gsem0, ssem0)

        return 0

    lax.fori_loop(0, _NCH, chunk, 0)
    for q in range(2):
        dummy = dstv[pl.ds(q * 16, 16)]
        pltpu.make_async_copy(rows.at[pl.ds(q * 16, 16)],
                              acc.at[dummy], ssem0).wait()
    plsc.subcore_barrier()

    def ochunk(k, _):
        cid = s + _NS * k

        @pl.when(cid < _NOCH)
        def _():
            sl = pl.ds(cid * _OCH, _OCH)

            @pl.when(c == 0)
            def _():
                pltpu.sync_copy(acc.at[sl], out0.at[sl])

            @pl.when(c == 1)
            def _():
                pltpu.sync_copy(acc.at[sl], out1.at[sl])

        return 0

    lax.fori_loop(0, _NOCH // _NS + 1, ochunk, 0)


_GRID = 10
_RBLK = _N // _GRID
_IBLK = _E // _D // _GRID


def _dot(a, b, dims):
    return lax.dot_general(a, b, (dims, ((), ())),
                           precision=lax.Precision.HIGHEST,
                           preferred_element_type=_f32)


def _half_map(i):
    return (lax.rem(i, _GRID // 2), 0)


def _tc_prep_body(dega_ref, degb_ref, feats_ref, src_ref, dst_ref,
                  dis_ref, gst_ref, gi1_ref, df_ref):
    deg = dega_ref[...] + degb_ref[...]
    safe = jnp.where(deg > 0, deg, 1.0)
    dis = jnp.where(deg > 0, lax.rsqrt(safe), 0.0)
    dis_ref[...] = dis
    g = feats_ref[...] * dis
    gst_ref[0] = g
    gst_ref[1] = -g

    @pl.when(pl.program_id(0) == 0)
    def _():
        srci = src_ref[...]
        dsti = dst_ref[...]
        hi = dsti >= _HALF
        gi1_ref[...] = jnp.where(hi, srci + _N, srci)
        df_ref[...] = jnp.where(hi, dsti - _HALF, dsti)


_tc_prep = pl.pallas_call(
    _tc_prep_body,
    grid=(_GRID,),
    in_specs=[
        pl.BlockSpec((_RBLK, 1), lambda i: (i, 0)),
        pl.BlockSpec((_RBLK, 1), lambda i: (i, 0)),
        pl.BlockSpec((_RBLK, _D), lambda i: (i, 0)),
        pl.BlockSpec((_E // _D, _D), lambda i: (0, 0)),
        pl.BlockSpec((_E // _D, _D), lambda i: (0, 0)),
    ],
    out_specs=[
        pl.BlockSpec((_RBLK, 1), lambda i: (i, 0)),
        pl.BlockSpec((2, _RBLK, _D), lambda i: (0, i, 0)),
        pl.BlockSpec((_E // _D, _D), lambda i: (0, 0)),
        pl.BlockSpec((_E // _D, _D), lambda i: (0, 0)),
    ],
    out_shape=[
        jax.ShapeDtypeStruct((_N, 1), _f32),
        jax.ShapeDtypeStruct((2, _N, _D), _f32),
        jax.ShapeDtypeStruct((_E // _D, _D), jnp.int32),
        jax.ShapeDtypeStruct((_E // _D, _D), jnp.int32),
    ],
)


def _fold_decode(a0, a1, i):
    sign = jnp.where(i < _GRID // 2, 0.5, -0.5)
    return 0.5 * a0 + sign * a1


def _tc_mid_body(a0_ref, a1_ref, dis_ref, feats_ref, w0_ref, w1_ref,
                 out01_ref, gst1_ref):
    i = pl.program_id(0)
    s0 = _fold_decode(a0_ref[...], a1_ref[...], i)
    dis = dis_ref[...]
    tx1 = -dis * s0
    out01_ref[...] = (_dot(feats_ref[...], w0_ref[...], ((1,), (0,)))
                      + _dot(tx1, w1_ref[...], ((1,), (0,))))
    g1 = dis * tx1
    gst1_ref[0] = g1
    gst1_ref[1] = -g1


_tc_mid = pl.pallas_call(
    _tc_mid_body,
    grid=(_GRID,),
    in_specs=[
        pl.BlockSpec((_RBLK, _D), _half_map),
        pl.BlockSpec((_RBLK, _D), _half_map),
        pl.BlockSpec((_RBLK, 1), lambda i: (i, 0)),
        pl.BlockSpec((_RBLK, _D), lambda i: (i, 0)),
        pl.BlockSpec((_D, _D), lambda i: (0, 0)),
        pl.BlockSpec((_D, _D), lambda i: (0, 0)),
    ],
    out_specs=[
        pl.BlockSpec((_RBLK, _D), lambda i: (i, 0)),
        pl.BlockSpec((2, _RBLK, _D), lambda i: (0, i, 0)),
    ],
    out_shape=[
        jax.ShapeDtypeStruct((_N, _D), _f32),
        jax.ShapeDtypeStruct((2, _N, _D), _f32),
    ],
)


def _tc_final_body(a0_ref, a1_ref, dis_ref, feats_ref, out01_ref, batch_ref,
                   w2_ref, bch_ref, gnw_ref, gnb_ref, gnms_ref, ow_ref, ob_ref,
                   out_ref, sums1, sums2, cnt, maxz):
    i = pl.program_id(0)

    @pl.when(i == 0)
    def _():
        sums1[...] = jnp.zeros_like(sums1)
        sums2[...] = jnp.zeros_like(sums2)
        cnt[...] = jnp.zeros_like(cnt)
        maxz[...] = jnp.full_like(maxz, -3.0e38)

    s1 = _fold_decode(a0_ref[...], a1_ref[...], i)
    tx2 = -2.0 * dis_ref[...] * s1 - feats_ref[...]
    x1 = out01_ref[...] + _dot(tx2, w2_ref[...], ((1,), (0,))) + bch_ref[...]
    x1 = jnp.clip(x1, -1.0, 1.0)

    bcol = batch_ref[...]
    iota = lax.broadcasted_iota(jnp.int32, (1, _B), 1).astype(_f32)
    onehot = (bcol == iota).astype(_f32)
    cnt[...] += _dot(onehot, jnp.ones((_RBLK, 1), _f32), ((0,), (0,)))
    sums1[...] += _dot(onehot, x1, ((0,), (0,)))
    sums2[...] += _dot(onehot, x1 * x1, ((0,), (0,)))

    sign = jnp.where(gnw_ref[...] >= 0, 1.0, -1.0)
    z = x1 * sign
    for b in range(_B):
        mb = jnp.max(jnp.where(bcol == float(b), z, -3.0e38), axis=0)
        maxz[b, :] = jnp.maximum(maxz[b, :], mb)

    @pl.when(i == _GRID - 1)
    def _():
        cn = cnt[...]
        m1 = sums1[...] / cn
        m2 = sums2[...] / cn
        ms = gnms_ref[...]
        w = gnw_ref[...]
        var = m2 - (2.0 * ms - ms * ms) * m1 * m1
        std = jnp.sqrt(var + _EPS)
        pooled = (jnp.abs(w) * maxz[...] - w * ms * m1) / std + gnb_ref[...]
        out_ref[...] = _dot(pooled, ow_ref[...], ((1,), (1,))) + ob_ref[...]


_tc_final = pl.pallas_call(
    _tc_final_body,
    grid=(_GRID,),
    in_specs=[
        pl.BlockSpec((_RBLK, _D), _half_map),
        pl.BlockSpec((_RBLK, _D), _half_map),
        pl.BlockSpec((_RBLK, 1), lambda i: (i, 0)),
        pl.BlockSpec((_RBLK, _D), lambda i: (i, 0)),
        pl.BlockSpec((_RBLK, _D), lambda i: (i, 0)),
        pl.BlockSpec((_RBLK, 1), lambda i: (i, 0)),
        pl.BlockSpec((_D, _D), lambda i: (0, 0)),
        pl.BlockSpec((1, _D), lambda i: (0, 0)),
        pl.BlockSpec((1, _D), lambda i: (0, 0)),
        pl.BlockSpec((1, _D), lambda i: (0, 0)),
        pl.BlockSpec((1, _D), lambda i: (0, 0)),
        pl.BlockSpec((_D, _D), lambda i: (0, 0)),
        pl.BlockSpec((1, _D), lambda i: (0, 0)),
    ],
    out_specs=pl.BlockSpec((_B, _D), lambda i: (0, 0)),
    out_shape=jax.ShapeDtypeStruct((_B, _D), _f32),
    scratch_shapes=[
        pltpu.VMEM((_B, _D), _f32),
        pltpu.VMEM((_B, _D), _f32),
        pltpu.VMEM((_B, 1), _f32),
        pltpu.VMEM((_B, _D), _f32),
    ],
)


def kernel(x, edge_index, W_cheb, b_cheb, gn_weight, gn_bias, gn_mean_scale,
           out_W, out_b):
    feats = x[:, :_D]
    batchf = x[:, -1:]
    src = edge_index[0]
    dst = edge_index[1]
    srcd = src.reshape(_NW, _DNCH, _DCH)
    src2 = src.reshape(_E // _D, _D)
    dst2 = dst.reshape(_E // _D, _D)
    gi0 = src

    dega, degb = _sc_deg(srcd)
    dis, gst0, gi1, df = _tc_prep(dega.reshape(_N, 1), degb.reshape(_N, 1),
                                  feats, src2, dst2)
    gi1 = gi1.reshape(_E)
    df = df.reshape(_E)

    a0, a1 = _sc_scatter(gst0.reshape(2 * _N, _D), gi0, gi1, df)
    out01, gst1 = _tc_mid(a0, a1, dis, feats, W_cheb[0], W_cheb[1])
    b0, b1 = _sc_scatter(gst1.reshape(2 * _N, _D), gi0, gi1, df)
    row = lambda v: v.reshape(1, _D)
    return _tc_final(b0, b1, dis, feats, out01, batchf, W_cheb[2],
                     row(b_cheb), row(gn_weight), row(gn_bias),
                     row(gn_mean_scale), out_W, row(out_b))


# speedup vs baseline: 9.8813x; 1.0397x over previous
import functools

import jax
import jax.numpy as jnp
from jax import lax
from jax.experimental import pallas as pl
from jax.experimental.pallas import tpu as pltpu
from jax.experimental.pallas import tpu_sc as plsc

_N = 10000
_E = 320000
_D = 128
_B = 16
_EPS = 1e-5

_NC = 2
_NS = 16
_HALF = _N // 2

_EPT = _E // _NS
_CH = 32
_NCH = _EPT // _CH
_IB = 800
_CPB = _IB // _CH
_NSLAB = _EPT // _IB
_ACC_R = 5008
_ZCH = _ACC_R // 16
_OCH = 40
_NOCH = _HALF // _OCH

_NW = _NC * _NS
_EPW = _E // _NW
_DCH = 80
_DNCH = _EPW // _DCH
_NZCH = _N // _DCH

_sc_mesh = plsc.VectorSubcoreMesh(core_axis_name="c", subcore_axis_name="s")

_f32 = jnp.float32


def _zero_vmem_2d(ref, nrows):
    zer = jnp.zeros((16,), _f32)

    def body(r, _):
        for cc in range(_D // 16):
            ref[r, pl.ds(cc * 16, 16)] = zer
        return 0

    lax.fori_loop(0, nrows, body, 0)


def _fill_vmem_1d(ref, n, value):
    for i in range(n // 16):
        ref[pl.ds(i * 16, 16)] = jnp.full((16,), value, _f32)


@functools.partial(
    pl.kernel,
    out_type=(
        jax.ShapeDtypeStruct((_N,), _f32),
        jax.ShapeDtypeStruct((_N,), _f32),
    ),
    mesh=_sc_mesh,
    scratch_types=[
        pltpu.VMEM((_DNCH, _DCH), jnp.int32),
        pltpu.VMEM((_DCH,), _f32),
        pltpu.VMEM((_DCH,), _f32),
        pltpu.VMEM_SHARED((_N,), _f32),
        pltpu.SemaphoreType.DMA,
    ],
)
def _sc_deg(src_hbm, out0, out1, srcv, onesv, zbuf, acc, dsem):
    c = lax.axis_index("c")
    s = lax.axis_index("s")
    wid = c * _NS + s
    pltpu.sync_copy(src_hbm.at[wid], srcv)
    _fill_vmem_1d(onesv, _DCH, 1.0)
    _fill_vmem_1d(zbuf, _DCH, 0.0)

    def zb(k, _):
        cid = s + _NS * k

        @pl.when(cid < _NZCH)
        def _():
            pltpu.sync_copy(zbuf, acc.at[pl.ds(cid * _DCH, _DCH)])

        return 0

    lax.fori_loop(0, _NZCH // _NS + 1, zb, 0)
    plsc.subcore_barrier()

    def body(j, _):
        pltpu.async_copy(onesv, acc.at[srcv.at[j]], sem=dsem, add=True)

        @pl.when(j >= 8)
        def _():
            pltpu.make_async_copy(onesv, acc.at[srcv.at[0]], dsem).wait()

        return 0

    lax.fori_loop(0, _DNCH, body, 0)
    for _ in range(8):
        pltpu.make_async_copy(onesv, acc.at[srcv.at[0]], dsem).wait()
    plsc.subcore_barrier()

    def cp(k, _):
        cid = s + _NS * k

        @pl.when(cid < _NZCH)
        def _():
            sl = pl.ds(cid * _DCH, _DCH)
            pltpu.sync_copy(acc.at[sl], zbuf)

            @pl.when(c == 0)
            def _():
                pltpu.sync_copy(zbuf, out0.at[sl])

            @pl.when(c == 1)
            def _():
                pltpu.sync_copy(zbuf, out1.at[sl])

        return 0

    lax.fori_loop(0, _NZCH // _NS + 1, cp, 0)


@functools.partial(
    pl.kernel,
    out_type=(
        jax.ShapeDtypeStruct((_HALF, _D), _f32),
        jax.ShapeDtypeStruct((_HALF, _D), _f32),
    ),
    mesh=_sc_mesh,
    scratch_types=[
        pltpu.VMEM((2 * _IB,), jnp.int32),
        pltpu.VMEM((2 * _IB,), jnp.int32),
        pltpu.VMEM((2 * _CH, _D), _f32),
        pltpu.VMEM_SHARED((_ACC_R, _D), _f32),
        pltpu.SemaphoreType.DMA,
        pltpu.SemaphoreType.DMA,
        pltpu.SemaphoreType.DMA,
        pltpu.SemaphoreType.DMA,
        pltpu.SemaphoreType.DMA,
    ],
)
def _sc_scatter(gst_hbm, gi0_hbm, gi1_hbm, df_hbm, out0, out1, srcv, dstv,
                rows, acc, gsem0, gsem1, ssem0, ssem1, isem):
    c = lax.axis_index("c")
    s = lax.axis_index("s")

    _zero_vmem_2d(rows, 16)

    def zchunk(k, _):
        cid = s + _NS * k

        @pl.when(cid < _ZCH)
        def _():
            pltpu.sync_copy(rows.at[pl.ds(0, 16)], acc.at[pl.ds(cid * 16, 16)])

        return 0

    lax.fori_loop(0, _ZCH // _NS + 1, zchunk, 0)
    plsc.subcore_barrier()

    base = s * _EPT

    def load_slab(m, half, sem):
        off = base + m * _IB
        dst_sl = pl.ds(half * _IB, _IB)
        if sem is None:
            @pl.when(c == 0)
            def _():
                pltpu.sync_copy(gi0_hbm.at[pl.ds(off, _IB)], srcv.at[dst_sl])

            @pl.when(c == 1)
            def _():
                pltpu.sync_copy(gi1_hbm.at[pl.ds(off, _IB)], srcv.at[dst_sl])

            pltpu.sync_copy(df_hbm.at[pl.ds(off, _IB)], dstv.at[dst_sl])
        else:
            @pl.when(c == 0)
            def _():
                pltpu.async_copy(gi0_hbm.at[pl.ds(off, _IB)], srcv.at[dst_sl],
                                 sem)

            @pl.when(c == 1)
            def _():
                pltpu.async_copy(gi1_hbm.at[pl.ds(off, _IB)], srcv.at[dst_sl],
                                 sem)

            pltpu.async_copy(df_hbm.at[pl.ds(off, _IB)], dstv.at[dst_sl], isem)

    load_slab(0, 0, None)
    load_slab(1, 1, isem)
    pltpu.async_copy(gst_hbm.at[srcv.at[pl.ds(0, _CH)]],
                     rows.at[pl.ds(0, _CH)], gsem0)

    def slab(m, _):
        half = lax.rem(m, 2) * _IB
        nhalf = _IB - half

        @pl.when((m >= 1) & (m + 1 < _NSLAB))
        def _():
            load_slab(m + 1, lax.rem(m + 1, 2), isem)

        def chunk(pos, _):
            sb = half + pos * _CH
            sb1 = jnp.where(pos + 1 < _CPB, sb + _CH, nhalf)
            last = (m + 1 >= _NSLAB) & (pos + 1 >= _CPB)

            @pl.when((pos + 1 == _CPB) & (m + 1 < _NSLAB))
            def _():
                pltpu.make_async_copy(df_hbm.at[pl.ds(base, 2 * _IB)],
                                      srcv, isem).wait()

            def work(roff, gsem, ssem, roff_n, gsem_n, ssem_n):
                @pl.when((m > 0) | (pos > 0))
                def _():
                    pltpu.make_async_copy(
                        gst_hbm.at[pl.ds(0, _CH)],
                        rows.at[pl.ds(roff_n, _CH)], ssem_n).wait()

                @pl.when(~last)
                def _():
                    pltpu.async_copy(gst_hbm.at[srcv.at[pl.ds(sb1, _CH)]],
                                     rows.at[pl.ds(roff_n, _CH)], gsem_n)

                pltpu.make_async_copy(gst_hbm.at[srcv.at[pl.ds(sb, _CH)]],
                                      rows.at[pl.ds(roff, _CH)], gsem).wait()
                for q in range(2):
                    idx16 = dstv[pl.ds(sb + q * 16, 16)]
                    pltpu.async_copy(rows.at[pl.ds(roff + q * 16, 16)],
                                     acc.at[idx16], ssem, add=True)

            par = lax.rem(m + pos, 2)

            @pl.when(par == 0)
            def _():
                work(0, gsem0, ssem0, _CH, gsem1, ssem1)

            @pl.when(par == 1)
            def _():
                work(_CH, gsem1, ssem1, 0, gsem0, ssem0)

            return 0

        lax.fori_loop(0, _CPB, chunk, 0)
        return 0

    lax.fori_loop(0, _NSLAB, slab, 0)
    pltpu.make_async_copy(gst_hbm.at[pl.ds(0, _CH)],
                          rows.at[pl.ds(0, _CH)], ssem0).wait()
    plsc.subcore_barrier()

    def ochunk(k, _):
        cid = s + _NS * k

        @pl.when(cid < _NOCH)
        def _():
            sl = pl.ds(cid * _OCH, _OCH)

            @pl.when(c == 0)
            def _():
                pltpu.sync_copy(acc.at[sl], out0.at[sl])

            @pl.when(c == 1)
            def _():
                pltpu.sync_copy(acc.at[sl], out1.at[sl])

        return 0

    lax.fori_loop(0, _NOCH // _NS + 1, ochunk, 0)


_GRID = 10
_RBLK = _N // _GRID
_IBLK = _E // _D // _GRID


def _dot(a, b, dims):
    return lax.dot_general(a, b, (dims, ((), ())),
                           precision=lax.Precision.HIGHEST,
                           preferred_element_type=_f32)


def _half_map(i):
    return (lax.rem(i, _GRID // 2), 0)


def _tc_prep_body(dega_ref, degb_ref, feats_ref, dis_ref, gst_ref):
    deg = dega_ref[...] + degb_ref[...]
    safe = jnp.where(deg > 0, deg, 1.0)
    dis = jnp.where(deg > 0, lax.rsqrt(safe), 0.0)
    dis_ref[...] = dis
    g = feats_ref[...] * dis
    gst_ref[0] = g
    gst_ref[1] = -g


_tc_prep = pl.pallas_call(
    _tc_prep_body,
    grid=(_GRID,),
    in_specs=[
        pl.BlockSpec((_RBLK, 1), lambda i: (i, 0)),
        pl.BlockSpec((_RBLK, 1), lambda i: (i, 0)),
        pl.BlockSpec((_RBLK, _D), lambda i: (i, 0)),
    ],
    out_specs=[
        pl.BlockSpec((_RBLK, 1), lambda i: (i, 0)),
        pl.BlockSpec((2, _RBLK, _D), lambda i: (0, i, 0)),
    ],
    out_shape=[
        jax.ShapeDtypeStruct((_N, 1), _f32),
        jax.ShapeDtypeStruct((2, _N, _D), _f32),
    ],
)


def _tc_idx_body(src_ref, dst_ref, gi1_ref, df_ref):
    srci = src_ref[...]
    dsti = dst_ref[...]
    hi = dsti >= _HALF
    gi1_ref[...] = jnp.where(hi, srci + _N, srci)
    df_ref[...] = jnp.where(hi, dsti - _HALF, dsti)


_tc_idx = pl.pallas_call(
    _tc_idx_body,
    out_shape=[
        jax.ShapeDtypeStruct((_E // _D, _D), jnp.int32),
        jax.ShapeDtypeStruct((_E // _D, _D), jnp.int32),
    ],
)


def _fold_decode(a0, a1, i):
    sign = jnp.where(i < _GRID // 2, 0.5, -0.5)
    return 0.5 * a0 + sign * a1


def _tc_mid_body(a0_ref, a1_ref, dis_ref, feats_ref, w0_ref, w1_ref,
                 out01_ref, gst1_ref):
    i = pl.program_id(0)
    s0 = _fold_decode(a0_ref[...], a1_ref[...], i)
    dis = dis_ref[...]
    tx1 = -dis * s0
    out01_ref[...] = (_dot(feats_ref[...], w0_ref[...], ((1,), (0,)))
                      + _dot(tx1, w1_ref[...], ((1,), (0,))))
    g1 = dis * tx1
    gst1_ref[0] = g1
    gst1_ref[1] = -g1


_tc_mid = pl.pallas_call(
    _tc_mid_body,
    grid=(_GRID,),
    in_specs=[
        pl.BlockSpec((_RBLK, _D), _half_map),
        pl.BlockSpec((_RBLK, _D), _half_map),
        pl.BlockSpec((_RBLK, 1), lambda i: (i, 0)),
        pl.BlockSpec((_RBLK, _D), lambda i: (i, 0)),
        pl.BlockSpec((_D, _D), lambda i: (0, 0)),
        pl.BlockSpec((_D, _D), lambda i: (0, 0)),
    ],
    out_specs=[
        pl.BlockSpec((_RBLK, _D), lambda i: (i, 0)),
        pl.BlockSpec((2, _RBLK, _D), lambda i: (0, i, 0)),
    ],
    out_shape=[
        jax.ShapeDtypeStruct((_N, _D), _f32),
        jax.ShapeDtypeStruct((2, _N, _D), _f32),
    ],
)


def _tc_final_body(a0_ref, a1_ref, dis_ref, feats_ref, out01_ref, batch_ref,
                   w2_ref, bch_ref, gnw_ref, gnb_ref, gnms_ref, ow_ref, ob_ref,
                   out_ref, sums1, sums2, cnt, maxz):
    i = pl.program_id(0)

    @pl.when(i == 0)
    def _():
        sums1[...] = jnp.zeros_like(sums1)
        sums2[...] = jnp.zeros_like(sums2)
        cnt[...] = jnp.zeros_like(cnt)
        maxz[...] = jnp.full_like(maxz, -3.0e38)

    s1 = _fold_decode(a0_ref[...], a1_ref[...], i)
    tx2 = -2.0 * dis_ref[...] * s1 - feats_ref[...]
    x1 = out01_ref[...] + _dot(tx2, w2_ref[...], ((1,), (0,))) + bch_ref[...]
    x1 = jnp.clip(x1, -1.0, 1.0)

    bcol = batch_ref[...]
    iota = lax.broadcasted_iota(jnp.int32, (1, _B), 1).astype(_f32)
    onehot = (bcol == iota).astype(_f32)
    cnt[...] += _dot(onehot, jnp.ones((_RBLK, 1), _f32), ((0,), (0,)))
    sums1[...] += _dot(onehot, x1, ((0,), (0,)))
    sums2[...] += _dot(onehot, x1 * x1, ((0,), (0,)))

    sign = jnp.where(gnw_ref[...] >= 0, 1.0, -1.0)
    z = x1 * sign
    for b in range(_B):
        mb = jnp.max(jnp.where(bcol == float(b), z, -3.0e38), axis=0)
        maxz[b, :] = jnp.maximum(maxz[b, :], mb)

    @pl.when(i == _GRID - 1)
    def _():
        cn = cnt[...]
        m1 = sums1[...] / cn
        m2 = sums2[...] / cn
        ms = gnms_ref[...]
        w = gnw_ref[...]
        var = m2 - (2.0 * ms - ms * ms) * m1 * m1
        std = jnp.sqrt(var + _EPS)
        pooled = (jnp.abs(w) * maxz[...] - w * ms * m1) / std + gnb_ref[...]
        out_ref[...] = _dot(pooled, ow_ref[...], ((1,), (1,))) + ob_ref[...]


_tc_final = pl.pallas_call(
    _tc_final_body,
    grid=(_GRID,),
    in_specs=[
        pl.BlockSpec((_RBLK, _D), _half_map),
        pl.BlockSpec((_RBLK, _D), _half_map),
        pl.BlockSpec((_RBLK, 1), lambda i: (i, 0)),
        pl.BlockSpec((_RBLK, _D), lambda i: (i, 0)),
        pl.BlockSpec((_RBLK, _D), lambda i: (i, 0)),
        pl.BlockSpec((_RBLK, 1), lambda i: (i, 0)),
        pl.BlockSpec((_D, _D), lambda i: (0, 0)),
        pl.BlockSpec((1, _D), lambda i: (0, 0)),
        pl.BlockSpec((1, _D), lambda i: (0, 0)),
        pl.BlockSpec((1, _D), lambda i: (0, 0)),
        pl.BlockSpec((1, _D), lambda i: (0, 0)),
        pl.BlockSpec((_D, _D), lambda i: (0, 0)),
        pl.BlockSpec((1, _D), lambda i: (0, 0)),
    ],
    out_specs=pl.BlockSpec((_B, _D), lambda i: (0, 0)),
    out_shape=jax.ShapeDtypeStruct((_B, _D), _f32),
    scratch_shapes=[
        pltpu.VMEM((_B, _D), _f32),
        pltpu.VMEM((_B, _D), _f32),
        pltpu.VMEM((_B, 1), _f32),
        pltpu.VMEM((_B, _D), _f32),
    ],
)


def kernel(x, edge_index, W_cheb, b_cheb, gn_weight, gn_bias, gn_mean_scale,
           out_W, out_b):
    feats = x[:, :_D]
    batchf = x[:, -1:]
    src = edge_index[0]
    dst = edge_index[1]
    srcd = src.reshape(_NW, _DNCH, _DCH)
    src2 = src.reshape(_E // _D, _D)
    dst2 = dst.reshape(_E // _D, _D)
    gi0 = src

    dega, degb = _sc_deg(srcd)
    gi1, df = _tc_idx(src2, dst2)
    dis, gst0 = _tc_prep(dega.reshape(_N, 1), degb.reshape(_N, 1), feats)
    gi1 = gi1.reshape(_E)
    df = df.reshape(_E)

    a0, a1 = _sc_scatter(gst0.reshape(2 * _N, _D), gi0, gi1, df)
    out01, gst1 = _tc_mid(a0, a1, dis, feats, W_cheb[0], W_cheb[1])
    b0, b1 = _sc_scatter(gst1.reshape(2 * _N, _D), gi0, gi1, df)
    row = lambda v: v.reshape(1, _D)
    return _tc_final(b0, b1, dis, feats, out01, batchf, W_cheb[2],
                     row(b_cheb), row(gn_weight), row(gn_bias),
                     row(gn_mean_scale), out_W, row(out_b))


# speedup vs baseline: 13.1390x; 1.3297x over previous
import functools

import jax
import jax.numpy as jnp
from jax import lax
from jax.experimental import pallas as pl
from jax.experimental.pallas import tpu as pltpu
from jax.experimental.pallas import tpu_sc as plsc

_N = 10000
_E = 320000
_D = 128
_B = 16
_EPS = 1e-5

_NC = 2
_NS = 16
_HALF = _N // 2

_EPT = _E // _NS
_CH = 32
_NCH = _EPT // _CH
_IB = 800
_CPB = _IB // _CH
_NSLAB = _EPT // _IB
_ACC_R = 5008
_ZCH = _ACC_R // 16
_OCH = 40
_NOCH = _HALF // _OCH

_NW = _NC * _NS
_EPW = _E // _NW
_DCH = 80
_DNCH = _EPW // _DCH
_NZCH = _N // _DCH

_sc_mesh = plsc.VectorSubcoreMesh(core_axis_name="c", subcore_axis_name="s")

_f32 = jnp.float32


def _zero_vmem_2d(ref, nrows):
    zer = jnp.zeros((16,), _f32)

    def body(r, _):
        for cc in range(_D // 16):
            ref[r, pl.ds(cc * 16, 16)] = zer
        return 0

    lax.fori_loop(0, nrows, body, 0)


def _fill_vmem_1d(ref, n, value):
    for i in range(n // 16):
        ref[pl.ds(i * 16, 16)] = jnp.full((16,), value, _f32)


@functools.partial(
    pl.kernel,
    out_type=(
        jax.ShapeDtypeStruct((_N,), _f32),
        jax.ShapeDtypeStruct((_N,), _f32),
    ),
    mesh=_sc_mesh,
    scratch_types=[
        pltpu.VMEM((_DNCH, _DCH), jnp.int32),
        pltpu.VMEM((_DCH,), _f32),
        pltpu.VMEM((_DCH,), _f32),
        pltpu.VMEM_SHARED((_N,), _f32),
        pltpu.SemaphoreType.DMA,
    ],
)
def _sc_deg(src_hbm, out0, out1, srcv, onesv, zbuf, acc, dsem):
    c = lax.axis_index("c")
    s = lax.axis_index("s")
    wid = c * _NS + s
    pltpu.sync_copy(src_hbm.at[wid], srcv)
    _fill_vmem_1d(onesv, _DCH, 1.0)
    _fill_vmem_1d(zbuf, _DCH, 0.0)

    def zb(k, _):
        cid = s + _NS * k

        @pl.when(cid < _NZCH)
        def _():
            pltpu.sync_copy(zbuf, acc.at[pl.ds(cid * _DCH, _DCH)])

        return 0

    lax.fori_loop(0, _NZCH // _NS + 1, zb, 0)
    plsc.subcore_barrier()

    def body(j, _):
        pltpu.async_copy(onesv, acc.at[srcv.at[j]], sem=dsem, add=True)

        @pl.when(j >= 8)
        def _():
            pltpu.make_async_copy(onesv, acc.at[srcv.at[0]], dsem).wait()

        return 0

    lax.fori_loop(0, _DNCH, body, 0)
    for _ in range(8):
        pltpu.make_async_copy(onesv, acc.at[srcv.at[0]], dsem).wait()
    plsc.subcore_barrier()

    def cp(k, _):
        cid = s + _NS * k

        @pl.when(cid < _NZCH)
        def _():
            sl = pl.ds(cid * _DCH, _DCH)
            pltpu.sync_copy(acc.at[sl], zbuf)

            @pl.when(c == 0)
            def _():
                pltpu.sync_copy(zbuf, out0.at[sl])

            @pl.when(c == 1)
            def _():
                pltpu.sync_copy(zbuf, out1.at[sl])

        return 0

    lax.fori_loop(0, _NZCH // _NS + 1, cp, 0)


@functools.partial(
    pl.kernel,
    out_type=(
        jax.ShapeDtypeStruct((_HALF, _D), _f32),
        jax.ShapeDtypeStruct((_HALF, _D), _f32),
    ),
    mesh=_sc_mesh,
    scratch_types=[
        pltpu.VMEM((2 * _IB,), jnp.int32),
        pltpu.VMEM((2 * _IB,), jnp.int32),
        pltpu.VMEM((3 * _CH, _D), _f32),
        pltpu.VMEM_SHARED((_ACC_R, _D), _f32),
        pltpu.SemaphoreType.DMA,
        pltpu.SemaphoreType.DMA,
        pltpu.SemaphoreType.DMA,
        pltpu.SemaphoreType.DMA,
        pltpu.SemaphoreType.DMA,
        pltpu.SemaphoreType.DMA,
        pltpu.SemaphoreType.DMA,
    ],
)
def _sc_scatter(gst_hbm, gi0_hbm, gi1_hbm, df_hbm, out0, out1, srcv, dstv,
                rows, acc, gsem0, gsem1, gsem2, ssem0, ssem1, ssem2, isem):
    c = lax.axis_index("c")
    s = lax.axis_index("s")
    gsems = (gsem0, gsem1, gsem2)
    ssems = (ssem0, ssem1, ssem2)

    _zero_vmem_2d(rows, 16)

    def zchunk(k, _):
        cid = s + _NS * k

        @pl.when(cid < _ZCH)
        def _():
            pltpu.sync_copy(rows.at[pl.ds(0, 16)], acc.at[pl.ds(cid * 16, 16)])

        return 0

    lax.fori_loop(0, _ZCH // _NS + 1, zchunk, 0)
    plsc.subcore_barrier()

    base = s * _EPT

    def load_slab(m, half, sem):
        off = base + m * _IB
        dst_sl = pl.ds(half * _IB, _IB)
        if sem is None:
            @pl.when(c == 0)
            def _():
                pltpu.sync_copy(gi0_hbm.at[pl.ds(off, _IB)], srcv.at[dst_sl])

            @pl.when(c == 1)
            def _():
                pltpu.sync_copy(gi1_hbm.at[pl.ds(off, _IB)], srcv.at[dst_sl])

            pltpu.sync_copy(df_hbm.at[pl.ds(off, _IB)], dstv.at[dst_sl])
        else:
            @pl.when(c == 0)
            def _():
                pltpu.async_copy(gi0_hbm.at[pl.ds(off, _IB)], srcv.at[dst_sl],
                                 sem)

            @pl.when(c == 1)
            def _():
                pltpu.async_copy(gi1_hbm.at[pl.ds(off, _IB)], srcv.at[dst_sl],
                                 sem)

            pltpu.async_copy(df_hbm.at[pl.ds(off, _IB)], dstv.at[dst_sl], sem)

    load_slab(0, 0, None)
    load_slab(1, 1, isem)
    pltpu.async_copy(gst_hbm.at[srcv.at[pl.ds(0, _CH)]],
                     rows.at[pl.ds(0, _CH)], gsem0)
    pltpu.async_copy(gst_hbm.at[srcv.at[pl.ds(_CH, _CH)]],
                     rows.at[pl.ds(_CH, _CH)], gsem1)

    def slab(m, _):
        half = lax.rem(m, 2) * _IB
        nhalf = _IB - half

        @pl.when((m >= 1) & (m + 1 < _NSLAB))
        def _():
            load_slab(m + 1, lax.rem(m + 1, 2), isem)

        def chunk(pos, _):
            sb = half + pos * _CH
            p2 = pos + 2
            sb2 = jnp.where(p2 < _CPB, half + p2 * _CH,
                            nhalf + (p2 - _CPB) * _CH)
            notlast2 = (m + 1 < _NSLAB) | (p2 < _CPB)

            @pl.when((p2 == _CPB) & (m + 1 < _NSLAB))
            def _():
                pltpu.make_async_copy(df_hbm.at[pl.ds(base, 2 * _IB)],
                                      srcv, isem).wait()

            def work(slot, nslot):
                roff = slot * _CH
                roff_n = nslot * _CH

                @pl.when((m > 0) | (pos > 0))
                def _():
                    pltpu.make_async_copy(
                        gst_hbm.at[pl.ds(0, _CH)],
                        rows.at[pl.ds(roff_n, _CH)], ssems[nslot]).wait()

                @pl.when(notlast2)
                def _():
                    pltpu.async_copy(gst_hbm.at[srcv.at[pl.ds(sb2, _CH)]],
                                     rows.at[pl.ds(roff_n, _CH)],
                                     gsems[nslot])

                pltpu.make_async_copy(gst_hbm.at[srcv.at[pl.ds(sb, _CH)]],
                                      rows.at[pl.ds(roff, _CH)],
                                      gsems[slot]).wait()
                for q in range(2):
                    idx16 = dstv[pl.ds(sb + q * 16, 16)]
                    pltpu.async_copy(rows.at[pl.ds(roff + q * 16, 16)],
                                     acc.at[idx16], ssems[slot], add=True)

            par = lax.rem(m + pos, 3)
            for slot in range(3):
                @pl.when(par == slot)
                def _(slot=slot):
                    work(slot, (slot + 2) % 3)

            return 0

        lax.fori_loop(0, _CPB, chunk, 0)
        return 0

    lax.fori_loop(0, _NSLAB, slab, 0)
    pltpu.make_async_copy(gst_hbm.at[pl.ds(0, _CH)],
                          rows.at[pl.ds(0, _CH)], ssem0).wait()
    plsc.subcore_barrier()

    def ochunk(k, _):
        cid = s + _NS * k

        @pl.when(cid < _NOCH)
        def _():
            sl = pl.ds(cid * _OCH, _OCH)

            @pl.when(c == 0)
            def _():
                pltpu.sync_copy(acc.at[sl], out0.at[sl])

            @pl.when(c == 1)
            def _():
                pltpu.sync_copy(acc.at[sl], out1.at[sl])

        return 0

    lax.fori_loop(0, _NOCH // _NS + 1, ochunk, 0)


_GRID = 10
_RBLK = _N // _GRID
_IBLK = _E // _D // _GRID


def _dot(a, b, dims):
    return lax.dot_general(a, b, (dims, ((), ())),
                           precision=lax.Precision.HIGHEST,
                           preferred_element_type=_f32)


def _half_map(i):
    return (lax.rem(i, _GRID // 2), 0)


def _tc_prep_body(dega_ref, degb_ref, feats_ref, dis_ref, gst_ref):
    deg = dega_ref[...] + degb_ref[...]
    safe = jnp.where(deg > 0, deg, 1.0)
    dis = jnp.where(deg > 0, lax.rsqrt(safe), 0.0)
    dis_ref[...] = dis
    g = feats_ref[...] * dis
    gst_ref[0] = g
    gst_ref[1] = -g


_tc_prep = pl.pallas_call(
    _tc_prep_body,
    grid=(_GRID,),
    in_specs=[
        pl.BlockSpec((_RBLK, 1), lambda i: (i, 0)),
        pl.BlockSpec((_RBLK, 1), lambda i: (i, 0)),
        pl.BlockSpec((_RBLK, _D), lambda i: (i, 0)),
    ],
    out_specs=[
        pl.BlockSpec((_RBLK, 1), lambda i: (i, 0)),
        pl.BlockSpec((2, _RBLK, _D), lambda i: (0, i, 0)),
    ],
    out_shape=[
        jax.ShapeDtypeStruct((_N, 1), _f32),
        jax.ShapeDtypeStruct((2, _N, _D), _f32),
    ],
)


def _tc_idx_body(src_ref, dst_ref, gi1_ref, df_ref):
    srci = src_ref[...]
    dsti = dst_ref[...]
    hi = dsti >= _HALF
    gi1_ref[...] = jnp.where(hi, srci + _N, srci)
    df_ref[...] = jnp.where(hi, dsti - _HALF, dsti)


_tc_idx = pl.pallas_call(
    _tc_idx_body,
    out_shape=[
        jax.ShapeDtypeStruct((_E // _D, _D), jnp.int32),
        jax.ShapeDtypeStruct((_E // _D, _D), jnp.int32),
    ],
)


def _fold_decode(a0, a1, i):
    sign = jnp.where(i < _GRID // 2, 0.5, -0.5)
    return 0.5 * a0 + sign * a1


def _tc_mid_body(a0_ref, a1_ref, dis_ref, feats_ref, w0_ref, w1_ref,
                 out01_ref, gst1_ref):
    i = pl.program_id(0)
    s0 = _fold_decode(a0_ref[...], a1_ref[...], i)
    dis = dis_ref[...]
    tx1 = -dis * s0
    out01_ref[...] = (_dot(feats_ref[...], w0_ref[...], ((1,), (0,)))
                      + _dot(tx1, w1_ref[...], ((1,), (0,))))
    g1 = dis * tx1
    gst1_ref[0] = g1
    gst1_ref[1] = -g1


_tc_mid = pl.pallas_call(
    _tc_mid_body,
    grid=(_GRID,),
    in_specs=[
        pl.BlockSpec((_RBLK, _D), _half_map),
        pl.BlockSpec((_RBLK, _D), _half_map),
        pl.BlockSpec((_RBLK, 1), lambda i: (i, 0)),
        pl.BlockSpec((_RBLK, _D), lambda i: (i, 0)),
        pl.BlockSpec((_D, _D), lambda i: (0, 0)),
        pl.BlockSpec((_D, _D), lambda i: (0, 0)),
    ],
    out_specs=[
        pl.BlockSpec((_RBLK, _D), lambda i: (i, 0)),
        pl.BlockSpec((2, _RBLK, _D), lambda i: (0, i, 0)),
    ],
    out_shape=[
        jax.ShapeDtypeStruct((_N, _D), _f32),
        jax.ShapeDtypeStruct((2, _N, _D), _f32),
    ],
)


def _tc_final_body(a0_ref, a1_ref, dis_ref, feats_ref, out01_ref, batch_ref,
                   w2_ref, bch_ref, gnw_ref, gnb_ref, gnms_ref, ow_ref, ob_ref,
                   out_ref, sums1, sums2, cnt, maxz):
    i = pl.program_id(0)

    @pl.when(i == 0)
    def _():
        sums1[...] = jnp.zeros_like(sums1)
        sums2[...] = jnp.zeros_like(sums2)
        cnt[...] = jnp.zeros_like(cnt)
        maxz[...] = jnp.full_like(maxz, -3.0e38)

    s1 = _fold_decode(a0_ref[...], a1_ref[...], i)
    tx2 = -2.0 * dis_ref[...] * s1 - feats_ref[...]
    x1 = out01_ref[...] + _dot(tx2, w2_ref[...], ((1,), (0,))) + bch_ref[...]
    x1 = jnp.clip(x1, -1.0, 1.0)

    bcol = batch_ref[...]
    iota = lax.broadcasted_iota(jnp.int32, (1, _B), 1).astype(_f32)
    onehot = (bcol == iota).astype(_f32)
    cnt[...] += _dot(onehot, jnp.ones((_RBLK, 1), _f32), ((0,), (0,)))
    sums1[...] += _dot(onehot, x1, ((0,), (0,)))
    sums2[...] += _dot(onehot, x1 * x1, ((0,), (0,)))

    sign = jnp.where(gnw_ref[...] >= 0, 1.0, -1.0)
    z = x1 * sign
    for b in range(_B):
        mb = jnp.max(jnp.where(bcol == float(b), z, -3.0e38), axis=0)
        maxz[b, :] = jnp.maximum(maxz[b, :], mb)

    @pl.when(i == _GRID - 1)
    def _():
        cn = cnt[...]
        m1 = sums1[...] / cn
        m2 = sums2[...] / cn
        ms = gnms_ref[...]
        w = gnw_ref[...]
        var = m2 - (2.0 * ms - ms * ms) * m1 * m1
        std = jnp.sqrt(var + _EPS)
        pooled = (jnp.abs(w) * maxz[...] - w * ms * m1) / std + gnb_ref[...]
        out_ref[...] = _dot(pooled, ow_ref[...], ((1,), (1,))) + ob_ref[...]


_tc_final = pl.pallas_call(
    _tc_final_body,
    grid=(_GRID,),
    in_specs=[
        pl.BlockSpec((_RBLK, _D), _half_map),
        pl.BlockSpec((_RBLK, _D), _half_map),
        pl.BlockSpec((_RBLK, 1), lambda i: (i, 0)),
        pl.BlockSpec((_RBLK, _D), lambda i: (i, 0)),
        pl.BlockSpec((_RBLK, _D), lambda i: (i, 0)),
        pl.BlockSpec((_RBLK, 1), lambda i: (i, 0)),
        pl.BlockSpec((_D, _D), lambda i: (0, 0)),
        pl.BlockSpec((1, _D), lambda i: (0, 0)),
        pl.BlockSpec((1, _D), lambda i: (0, 0)),
        pl.BlockSpec((1, _D), lambda i: (0, 0)),
        pl.BlockSpec((1, _D), lambda i: (0, 0)),
        pl.BlockSpec((_D, _D), lambda i: (0, 0)),
        pl.BlockSpec((1, _D), lambda i: (0, 0)),
    ],
    out_specs=pl.BlockSpec((_B, _D), lambda i: (0, 0)),
    out_shape=jax.ShapeDtypeStruct((_B, _D), _f32),
    scratch_shapes=[
        pltpu.VMEM((_B, _D), _f32),
        pltpu.VMEM((_B, _D), _f32),
        pltpu.VMEM((_B, 1), _f32),
        pltpu.VMEM((_B, _D), _f32),
    ],
)


def kernel(x, edge_index, W_cheb, b_cheb, gn_weight, gn_bias, gn_mean_scale,
           out_W, out_b):
    feats = x[:, :_D]
    batchf = x[:, -1:]
    src = edge_index[0]
    dst = edge_index[1]
    srcd = src.reshape(_NW, _DNCH, _DCH)
    src2 = src.reshape(_E // _D, _D)
    dst2 = dst.reshape(_E // _D, _D)
    gi0 = src

    dega, degb = _sc_deg(srcd)
    gi1, df = _tc_idx(src2, dst2)
    dis, gst0 = _tc_prep(dega.reshape(_N, 1), degb.reshape(_N, 1), feats)
    gi1 = gi1.reshape(_E)
    df = df.reshape(_E)

    a0, a1 = _sc_scatter(gst0.reshape(2 * _N, _D), gi0, gi1, df)
    out01, gst1 = _tc_mid(a0, a1, dis, feats, W_cheb[0], W_cheb[1])
    b0, b1 = _sc_scatter(gst1.reshape(2 * _N, _D), gi0, gi1, df)
    row = lambda v: v.reshape(1, _D)
    return _tc_final(b0, b1, dis, feats, out01, batchf, W_cheb[2],
                     row(b_cheb), row(gn_weight), row(gn_bias),
                     row(gn_mean_scale), out_W, row(out_b))


# speedup vs baseline: 13.1745x; 1.0027x over previous
import functools

import jax
import jax.numpy as jnp
from jax import lax
from jax.experimental import pallas as pl
from jax.experimental.pallas import tpu as pltpu
from jax.experimental.pallas import tpu_sc as plsc

_N = 10000
_E = 320000
_D = 128
_B = 16
_EPS = 1e-5

_NC = 2
_NS = 16
_HALF = _N // 2

_EPT = _E // _NS
_CH = 32
_NCH = _EPT // _CH
_IB = 800
_CPB = _IB // _CH
_NSLAB = _EPT // _IB
_ACC_R = 5008
_ZCH = _ACC_R // 16
_OCH = 40
_NOCH = _HALF // _OCH

_NW = _NC * _NS
_EPW = _E // _NW
_DCH = 80
_DNCH = _EPW // _DCH
_NZCH = _N // _DCH

_sc_mesh = plsc.VectorSubcoreMesh(core_axis_name="c", subcore_axis_name="s")

_f32 = jnp.float32


def _zero_vmem_2d(ref, nrows):
    zer = jnp.zeros((16,), _f32)

    def body(r, _):
        for cc in range(_D // 16):
            ref[r, pl.ds(cc * 16, 16)] = zer
        return 0

    lax.fori_loop(0, nrows, body, 0)


def _fill_vmem_1d(ref, n, value):
    for i in range(n // 16):
        ref[pl.ds(i * 16, 16)] = jnp.full((16,), value, _f32)


@functools.partial(
    pl.kernel,
    out_type=(
        jax.ShapeDtypeStruct((_N,), _f32),
        jax.ShapeDtypeStruct((_N,), _f32),
    ),
    mesh=_sc_mesh,
    scratch_types=[
        pltpu.VMEM((_DNCH, _DCH), jnp.int32),
        pltpu.VMEM((_DCH,), _f32),
        pltpu.VMEM((_DCH,), _f32),
        pltpu.VMEM_SHARED((_N,), _f32),
        pltpu.SemaphoreType.DMA,
    ],
)
def _sc_deg(src_hbm, out0, out1, srcv, onesv, zbuf, acc, dsem):
    c = lax.axis_index("c")
    s = lax.axis_index("s")
    wid = c * _NS + s
    pltpu.sync_copy(src_hbm.at[wid], srcv)
    _fill_vmem_1d(onesv, _DCH, 1.0)
    _fill_vmem_1d(zbuf, _DCH, 0.0)

    def zb(k, _):
        cid = s + _NS * k

        @pl.when(cid < _NZCH)
        def _():
            pltpu.sync_copy(zbuf, acc.at[pl.ds(cid * _DCH, _DCH)])

        return 0

    lax.fori_loop(0, _NZCH // _NS + 1, zb, 0)
    plsc.subcore_barrier()

    def body(j, _):
        pltpu.async_copy(onesv, acc.at[srcv.at[j]], sem=dsem, add=True)

        @pl.when(j >= 8)
        def _():
            pltpu.make_async_copy(onesv, acc.at[srcv.at[0]], dsem).wait()

        return 0

    lax.fori_loop(0, _DNCH, body, 0)
    for _ in range(8):
        pltpu.make_async_copy(onesv, acc.at[srcv.at[0]], dsem).wait()
    plsc.subcore_barrier()

    def cp(k, _):
        cid = s + _NS * k

        @pl.when(cid < _NZCH)
        def _():
            sl = pl.ds(cid * _DCH, _DCH)
            pltpu.sync_copy(acc.at[sl], zbuf)

            @pl.when(c == 0)
            def _():
                pltpu.sync_copy(zbuf, out0.at[sl])

            @pl.when(c == 1)
            def _():
                pltpu.sync_copy(zbuf, out1.at[sl])

        return 0

    lax.fori_loop(0, _NZCH // _NS + 1, cp, 0)


@functools.partial(
    pl.kernel,
    out_type=(
        jax.ShapeDtypeStruct((_HALF, _D), _f32),
        jax.ShapeDtypeStruct((_HALF, _D), _f32),
    ),
    mesh=_sc_mesh,
    scratch_types=[
        pltpu.VMEM((2 * _IB,), jnp.int32),
        pltpu.VMEM((2 * _IB,), jnp.int32),
        pltpu.VMEM((3 * _CH, _D), _f32),
        pltpu.VMEM_SHARED((_ACC_R, _D), _f32),
        pltpu.SemaphoreType.DMA,
        pltpu.SemaphoreType.DMA,
        pltpu.SemaphoreType.DMA,
        pltpu.SemaphoreType.DMA,
        pltpu.SemaphoreType.DMA,
        pltpu.SemaphoreType.DMA,
        pltpu.SemaphoreType.DMA,
    ],
)
def _sc_scatter(gst_hbm, gi0_hbm, gi1_hbm, df_hbm, out0, out1, srcv, dstv,
                rows, acc, gsem0, gsem1, gsem2, ssem0, ssem1, ssem2, isem):
    c = lax.axis_index("c")
    s = lax.axis_index("s")
    gsems = (gsem0, gsem1, gsem2)
    ssems = (ssem0, ssem1, ssem2)

    base = s * _EPT

    def load_slab(m, half, sem):
        off = base + m * _IB
        dst_sl = pl.ds(half * _IB, _IB)
        if sem is None:
            @pl.when(c == 0)
            def _():
                pltpu.sync_copy(gi0_hbm.at[pl.ds(off, _IB)], srcv.at[dst_sl])

            @pl.when(c == 1)
            def _():
                pltpu.sync_copy(gi1_hbm.at[pl.ds(off, _IB)], srcv.at[dst_sl])

            pltpu.sync_copy(df_hbm.at[pl.ds(off, _IB)], dstv.at[dst_sl])
        else:
            @pl.when(c == 0)
            def _():
                pltpu.async_copy(gi0_hbm.at[pl.ds(off, _IB)], srcv.at[dst_sl],
                                 sem)

            @pl.when(c == 1)
            def _():
                pltpu.async_copy(gi1_hbm.at[pl.ds(off, _IB)], srcv.at[dst_sl],
                                 sem)

            pltpu.async_copy(df_hbm.at[pl.ds(off, _IB)], dstv.at[dst_sl], sem)

    load_slab(0, 0, isem)
    load_slab(1, 1, isem)

    _zero_vmem_2d(rows, 16)

    def zchunk(k, _):
        cid = s + _NS * k

        @pl.when(cid < _ZCH)
        def _():
            pltpu.sync_copy(rows.at[pl.ds(0, 16)], acc.at[pl.ds(cid * 16, 16)])

        return 0

    lax.fori_loop(0, _ZCH // _NS + 1, zchunk, 0)
    plsc.subcore_barrier()

    pltpu.make_async_copy(df_hbm.at[pl.ds(base, 4 * _IB)],
                          srcv, isem).wait()
    pltpu.async_copy(gst_hbm.at[srcv.at[pl.ds(0, _CH)]],
                     rows.at[pl.ds(0, _CH)], gsem0)
    pltpu.async_copy(gst_hbm.at[srcv.at[pl.ds(_CH, _CH)]],
                     rows.at[pl.ds(_CH, _CH)], gsem1)

    def slab(m, _):
        half = lax.rem(m, 2) * _IB
        nhalf = _IB - half

        @pl.when((m >= 1) & (m + 1 < _NSLAB))
        def _():
            load_slab(m + 1, lax.rem(m + 1, 2), isem)

        def chunk(pos, _):
            sb = half + pos * _CH
            p2 = pos + 2
            sb2 = jnp.where(p2 < _CPB, half + p2 * _CH,
                            nhalf + (p2 - _CPB) * _CH)
            notlast2 = (m + 1 < _NSLAB) | (p2 < _CPB)

            @pl.when((p2 == _CPB) & (m + 1 < _NSLAB))
            def _():
                pltpu.make_async_copy(df_hbm.at[pl.ds(base, 2 * _IB)],
                                      srcv, isem).wait()

            def work(slot, nslot):
                roff = slot * _CH
                roff_n = nslot * _CH

                @pl.when((m > 0) | (pos > 0))
                def _():
                    pltpu.make_async_copy(
                        gst_hbm.at[pl.ds(0, _CH)],
                        rows.at[pl.ds(roff_n, _CH)], ssems[nslot]).wait()

                @pl.when(notlast2)
                def _():
                    pltpu.async_copy(gst_hbm.at[srcv.at[pl.ds(sb2, _CH)]],
                                     rows.at[pl.ds(roff_n, _CH)],
                                     gsems[nslot])

                pltpu.make_async_copy(gst_hbm.at[srcv.at[pl.ds(sb, _CH)]],
                                      rows.at[pl.ds(roff, _CH)],
                                      gsems[slot]).wait()
                for q in range(2):
                    idx16 = dstv[pl.ds(sb + q * 16, 16)]
                    pltpu.async_copy(rows.at[pl.ds(roff + q * 16, 16)],
                                     acc.at[idx16], ssems[slot], add=True)

            par = lax.rem(m + pos, 3)
            for slot in range(3):
                @pl.when(par == slot)
                def _(slot=slot):
                    work(slot, (slot + 2) % 3)

            return 0

        lax.fori_loop(0, _CPB, chunk, 0)
        return 0

    lax.fori_loop(0, _NSLAB, slab, 0)
    pltpu.make_async_copy(gst_hbm.at[pl.ds(0, _CH)],
                          rows.at[pl.ds(0, _CH)], ssem0).wait()
    plsc.subcore_barrier()

    def ochunk(k, _):
        cid = s + _NS * k

        @pl.when(cid < _NOCH)
        def _():
            sl = pl.ds(cid * _OCH, _OCH)

            @pl.when(c == 0)
            def _():
                pltpu.sync_copy(acc.at[sl], out0.at[sl])

            @pl.when(c == 1)
            def _():
                pltpu.sync_copy(acc.at[sl], out1.at[sl])

        return 0

    lax.fori_loop(0, _NOCH // _NS + 1, ochunk, 0)


_GRID = 10
_RBLK = _N // _GRID
_IBLK = _E // _D // _GRID


def _dot(a, b, dims):
    return lax.dot_general(a, b, (dims, ((), ())),
                           precision=lax.Precision.HIGHEST,
                           preferred_element_type=_f32)


def _half_map(i):
    return (lax.rem(i, _GRID // 2), 0)


def _tc_prep_body(dega_ref, degb_ref, feats_ref, dis_ref, gst_ref):
    deg = dega_ref[...] + degb_ref[...]
    safe = jnp.where(deg > 0, deg, 1.0)
    dis = jnp.where(deg > 0, lax.rsqrt(safe), 0.0)
    dis_ref[...] = dis
    g = feats_ref[...] * dis
    gst_ref[0] = g
    gst_ref[1] = -g


_tc_prep = pl.pallas_call(
    _tc_prep_body,
    grid=(_GRID,),
    in_specs=[
        pl.BlockSpec((_RBLK, 1), lambda i: (i, 0)),
        pl.BlockSpec((_RBLK, 1), lambda i: (i, 0)),
        pl.BlockSpec((_RBLK, _D), lambda i: (i, 0)),
    ],
    out_specs=[
        pl.BlockSpec((_RBLK, 1), lambda i: (i, 0)),
        pl.BlockSpec((2, _RBLK, _D), lambda i: (0, i, 0)),
    ],
    out_shape=[
        jax.ShapeDtypeStruct((_N, 1), _f32),
        jax.ShapeDtypeStruct((2, _N, _D), _f32),
    ],
)


def _tc_idx_body(src_ref, dst_ref, gi1_ref, df_ref):
    srci = src_ref[...]
    dsti = dst_ref[...]
    hi = dsti >= _HALF
    gi1_ref[...] = jnp.where(hi, srci + _N, srci)
    df_ref[...] = jnp.where(hi, dsti - _HALF, dsti)


_tc_idx = pl.pallas_call(
    _tc_idx_body,
    out_shape=[
        jax.ShapeDtypeStruct((_E // _D, _D), jnp.int32),
        jax.ShapeDtypeStruct((_E // _D, _D), jnp.int32),
    ],
)


def _fold_decode(a0, a1, i):
    sign = jnp.where(i < _GRID // 2, 0.5, -0.5)
    return 0.5 * a0 + sign * a1


def _tc_mid_body(a0_ref, a1_ref, dis_ref, feats_ref, w0_ref, w1_ref,
                 out01_ref, gst1_ref):
    i = pl.program_id(0)
    s0 = _fold_decode(a0_ref[...], a1_ref[...], i)
    dis = dis_ref[...]
    tx1 = -dis * s0
    out01_ref[...] = (_dot(feats_ref[...], w0_ref[...], ((1,), (0,)))
                      + _dot(tx1, w1_ref[...], ((1,), (0,))))
    g1 = dis * tx1
    gst1_ref[0] = g1
    gst1_ref[1] = -g1


_tc_mid = pl.pallas_call(
    _tc_mid_body,
    grid=(_GRID,),
    in_specs=[
        pl.BlockSpec((_RBLK, _D), _half_map),
        pl.BlockSpec((_RBLK, _D), _half_map),
        pl.BlockSpec((_RBLK, 1), lambda i: (i, 0)),
        pl.BlockSpec((_RBLK, _D), lambda i: (i, 0)),
        pl.BlockSpec((_D, _D), lambda i: (0, 0)),
        pl.BlockSpec((_D, _D), lambda i: (0, 0)),
    ],
    out_specs=[
        pl.BlockSpec((_RBLK, _D), lambda i: (i, 0)),
        pl.BlockSpec((2, _RBLK, _D), lambda i: (0, i, 0)),
    ],
    out_shape=[
        jax.ShapeDtypeStruct((_N, _D), _f32),
        jax.ShapeDtypeStruct((2, _N, _D), _f32),
    ],
)


def _tc_final_body(a0_ref, a1_ref, dis_ref, feats_ref, out01_ref, batch_ref,
                   w2_ref, bch_ref, gnw_ref, gnb_ref, gnms_ref, ow_ref, ob_ref,
                   out_ref, sums1, sums2, cnt, maxz):
    i = pl.program_id(0)

    @pl.when(i == 0)
    def _():
        sums1[...] = jnp.zeros_like(sums1)
        sums2[...] = jnp.zeros_like(sums2)
        cnt[...] = jnp.zeros_like(cnt)
        maxz[...] = jnp.full_like(maxz, -3.0e38)

    s1 = _fold_decode(a0_ref[...], a1_ref[...], i)
    tx2 = -2.0 * dis_ref[...] * s1 - feats_ref[...]
    x1 = out01_ref[...] + _dot(tx2, w2_ref[...], ((1,), (0,))) + bch_ref[...]
    x1 = jnp.clip(x1, -1.0, 1.0)

    bcol = batch_ref[...]
    iota = lax.broadcasted_iota(jnp.int32, (1, _B), 1).astype(_f32)
    onehot = (bcol == iota).astype(_f32)
    cnt[...] += _dot(onehot, jnp.ones((_RBLK, 1), _f32), ((0,), (0,)))
    sums1[...] += _dot(onehot, x1, ((0,), (0,)))
    sums2[...] += _dot(onehot, x1 * x1, ((0,), (0,)))

    sign = jnp.where(gnw_ref[...] >= 0, 1.0, -1.0)
    z = x1 * sign
    for b in range(_B):
        mb = jnp.max(jnp.where(bcol == float(b), z, -3.0e38), axis=0)
        maxz[b, :] = jnp.maximum(maxz[b, :], mb)

    @pl.when(i == _GRID - 1)
    def _():
        cn = cnt[...]
        m1 = sums1[...] / cn
        m2 = sums2[...] / cn
        ms = gnms_ref[...]
        w = gnw_ref[...]
        var = m2 - (2.0 * ms - ms * ms) * m1 * m1
        std = jnp.sqrt(var + _EPS)
        pooled = (jnp.abs(w) * maxz[...] - w * ms * m1) / std + gnb_ref[...]
        out_ref[...] = _dot(pooled, ow_ref[...], ((1,), (1,))) + ob_ref[...]


_tc_final = pl.pallas_call(
    _tc_final_body,
    grid=(_GRID,),
    in_specs=[
        pl.BlockSpec((_RBLK, _D), _half_map),
        pl.BlockSpec((_RBLK, _D), _half_map),
        pl.BlockSpec((_RBLK, 1), lambda i: (i, 0)),
        pl.BlockSpec((_RBLK, _D), lambda i: (i, 0)),
        pl.BlockSpec((_RBLK, _D), lambda i: (i, 0)),
        pl.BlockSpec((_RBLK, 1), lambda i: (i, 0)),
        pl.BlockSpec((_D, _D), lambda i: (0, 0)),
        pl.BlockSpec((1, _D), lambda i: (0, 0)),
        pl.BlockSpec((1, _D), lambda i: (0, 0)),
        pl.BlockSpec((1, _D), lambda i: (0, 0)),
        pl.BlockSpec((1, _D), lambda i: (0, 0)),
        pl.BlockSpec((_D, _D), lambda i: (0, 0)),
        pl.BlockSpec((1, _D), lambda i: (0, 0)),
    ],
    out_specs=pl.BlockSpec((_B, _D), lambda i: (0, 0)),
    out_shape=jax.ShapeDtypeStruct((_B, _D), _f32),
    scratch_shapes=[
        pltpu.VMEM((_B, _D), _f32),
        pltpu.VMEM((_B, _D), _f32),
        pltpu.VMEM((_B, 1), _f32),
        pltpu.VMEM((_B, _D), _f32),
    ],
)


def kernel(x, edge_index, W_cheb, b_cheb, gn_weight, gn_bias, gn_mean_scale,
           out_W, out_b):
    feats = x[:, :_D]
    batchf = x[:, -1:]
    src = edge_index[0]
    dst = edge_index[1]
    srcd = src.reshape(_NW, _DNCH, _DCH)
    src2 = src.reshape(_E // _D, _D)
    dst2 = dst.reshape(_E // _D, _D)
    gi0 = src

    dega, degb = _sc_deg(srcd)
    gi1, df = _tc_idx(src2, dst2)
    dis, gst0 = _tc_prep(dega.reshape(_N, 1), degb.reshape(_N, 1), feats)
    gi1 = gi1.reshape(_E)
    df = df.reshape(_E)

    a0, a1 = _sc_scatter(gst0.reshape(2 * _N, _D), gi0, gi1, df)
    out01, gst1 = _tc_mid(a0, a1, dis, feats, W_cheb[0], W_cheb[1])
    b0, b1 = _sc_scatter(gst1.reshape(2 * _N, _D), gi0, gi1, df)
    row = lambda v: v.reshape(1, _D)
    return _tc_final(b0, b1, dis, feats, out01, batchf, W_cheb[2],
                     row(b_cheb), row(gn_weight), row(gn_bias),
                     row(gn_mean_scale), out_W, row(out_b))


# speedup vs baseline: 13.3102x; 1.0103x over previous
import functools

import jax
import jax.numpy as jnp
from jax import lax
from jax.experimental import pallas as pl
from jax.experimental.pallas import tpu as pltpu
from jax.experimental.pallas import tpu_sc as plsc

_N = 10000
_E = 320000
_D = 128
_B = 16
_EPS = 1e-5

_NC = 2
_NS = 16
_HALF = _N // 2

_EPT = _E // _NS
_CH = 32
_NCH = _EPT // _CH
_IB = 800
_CPB = _IB // _CH
_NSLAB = _EPT // _IB
_ACC_R = 5008
_ZCH = _ACC_R // 16
_OCH = 40
_NOCH = _HALF // _OCH

_NW = _NC * _NS
_EPW = _E // _NW
_DCH = 80
_DNCH = _EPW // _DCH
_NZCH = _N // _DCH

_sc_mesh = plsc.VectorSubcoreMesh(core_axis_name="c", subcore_axis_name="s")

_f32 = jnp.float32


def _zero_vmem_2d(ref, nrows):
    zer = jnp.zeros((16,), _f32)

    def body(r, _):
        for cc in range(_D // 16):
            ref[r, pl.ds(cc * 16, 16)] = zer
        return 0

    lax.fori_loop(0, nrows, body, 0)


def _fill_vmem_1d(ref, n, value):
    for i in range(n // 16):
        ref[pl.ds(i * 16, 16)] = jnp.full((16,), value, _f32)


@functools.partial(
    pl.kernel,
    out_type=(
        jax.ShapeDtypeStruct((_N,), _f32),
        jax.ShapeDtypeStruct((_N,), _f32),
    ),
    mesh=_sc_mesh,
    scratch_types=[
        pltpu.VMEM((_DNCH, _DCH), jnp.int32),
        pltpu.VMEM((_DCH,), _f32),
        pltpu.VMEM((_DCH,), _f32),
        pltpu.VMEM_SHARED((_N,), _f32),
        pltpu.SemaphoreType.DMA,
    ],
)
def _sc_deg(src_hbm, out0, out1, srcv, onesv, zbuf, acc, dsem):
    c = lax.axis_index("c")
    s = lax.axis_index("s")
    wid = c * _NS + s
    pltpu.sync_copy(src_hbm.at[wid], srcv)
    _fill_vmem_1d(onesv, _DCH, 1.0)
    _fill_vmem_1d(zbuf, _DCH, 0.0)

    def zb(k, _):
        cid = s + _NS * k

        @pl.when(cid < _NZCH)
        def _():
            pltpu.sync_copy(zbuf, acc.at[pl.ds(cid * _DCH, _DCH)])

        return 0

    lax.fori_loop(0, _NZCH // _NS + 1, zb, 0)
    plsc.subcore_barrier()

    def body(j, _):
        pltpu.async_copy(onesv, acc.at[srcv.at[j]], sem=dsem, add=True)

        @pl.when(j >= 8)
        def _():
            pltpu.make_async_copy(onesv, acc.at[srcv.at[0]], dsem).wait()

        return 0

    lax.fori_loop(0, _DNCH, body, 0)
    for _ in range(8):
        pltpu.make_async_copy(onesv, acc.at[srcv.at[0]], dsem).wait()
    plsc.subcore_barrier()

    def cp(k, _):
        cid = s + _NS * k

        @pl.when(cid < _NZCH)
        def _():
            sl = pl.ds(cid * _DCH, _DCH)
            pltpu.sync_copy(acc.at[sl], zbuf)

            @pl.when(c == 0)
            def _():
                pltpu.sync_copy(zbuf, out0.at[sl])

            @pl.when(c == 1)
            def _():
                pltpu.sync_copy(zbuf, out1.at[sl])

        return 0

    lax.fori_loop(0, _NZCH // _NS + 1, cp, 0)


@functools.partial(
    pl.kernel,
    out_type=(
        jax.ShapeDtypeStruct((_HALF, _D), _f32),
        jax.ShapeDtypeStruct((_HALF, _D), _f32),
    ),
    mesh=_sc_mesh,
    scratch_types=[
        pltpu.VMEM((2 * _IB,), jnp.int32),
        pltpu.VMEM((2 * _IB,), jnp.int32),
        pltpu.VMEM((3 * _CH, _D), _f32),
        pltpu.VMEM_SHARED((_ACC_R, _D), _f32),
        pltpu.SemaphoreType.DMA,
        pltpu.SemaphoreType.DMA,
        pltpu.SemaphoreType.DMA,
        pltpu.SemaphoreType.DMA,
        pltpu.SemaphoreType.DMA,
        pltpu.SemaphoreType.DMA,
        pltpu.SemaphoreType.DMA,
    ],
)
def _sc_scatter(gst_hbm, gi0_hbm, gi1_hbm, df_hbm, out0, out1, srcv, dstv,
                rows, acc, gsem0, gsem1, gsem2, ssem0, ssem1, ssem2, isem):
    c = lax.axis_index("c")
    s = lax.axis_index("s")
    gsems = (gsem0, gsem1, gsem2)
    ssems = (ssem0, ssem1, ssem2)

    base = s * _EPT

    def load_slab(m, half, sem):
        off = base + m * _IB
        dst_sl = pl.ds(half * _IB, _IB)
        if sem is None:
            @pl.when(c == 0)
            def _():
                pltpu.sync_copy(gi0_hbm.at[pl.ds(off, _IB)], srcv.at[dst_sl])

            @pl.when(c == 1)
            def _():
                pltpu.sync_copy(gi1_hbm.at[pl.ds(off, _IB)], srcv.at[dst_sl])

            pltpu.sync_copy(df_hbm.at[pl.ds(off, _IB)], dstv.at[dst_sl])
        else:
            @pl.when(c == 0)
            def _():
                pltpu.async_copy(gi0_hbm.at[pl.ds(off, _IB)], srcv.at[dst_sl],
                                 sem)

            @pl.when(c == 1)
            def _():
                pltpu.async_copy(gi1_hbm.at[pl.ds(off, _IB)], srcv.at[dst_sl],
                                 sem)

            pltpu.async_copy(df_hbm.at[pl.ds(off, _IB)], dstv.at[dst_sl], sem)

    load_slab(0, 0, isem)
    load_slab(1, 1, isem)

    _zero_vmem_2d(rows, 16)

    def zchunk(k, _):
        cid = s + _NS * k

        @pl.when(cid < _ZCH)
        def _():
            pltpu.sync_copy(rows.at[pl.ds(0, 16)], acc.at[pl.ds(cid * 16, 16)])

        return 0

    lax.fori_loop(0, _ZCH // _NS + 1, zchunk, 0)
    plsc.subcore_barrier()

    pltpu.make_async_copy(df_hbm.at[pl.ds(base, 4 * _IB)],
                          srcv, isem).wait()
    pltpu.async_copy(gst_hbm.at[srcv.at[pl.ds(0, _CH)]],
                     rows.at[pl.ds(0, _CH)], gsem0)
    pltpu.async_copy(gst_hbm.at[srcv.at[pl.ds(_CH, _CH)]],
                     rows.at[pl.ds(_CH, _CH)], gsem1)

    def slab(m, _):
        half = lax.rem(m, 2) * _IB
        nhalf = _IB - half

        @pl.when((m >= 1) & (m + 1 < _NSLAB))
        def _():
            load_slab(m + 1, lax.rem(m + 1, 2), isem)

        def chunk(pos, _):
            sb = half + pos * _CH
            p2 = pos + 2
            sb2 = jnp.where(p2 < _CPB, half + p2 * _CH,
                            nhalf + (p2 - _CPB) * _CH)
            notlast2 = (m + 1 < _NSLAB) | (p2 < _CPB)

            @pl.when((p2 == _CPB) & (m + 1 < _NSLAB))
            def _():
                pltpu.make_async_copy(df_hbm.at[pl.ds(base, 2 * _IB)],
                                      srcv, isem).wait()

            def work(slot, nslot):
                roff = slot * _CH
                roff_n = nslot * _CH

                @pl.when((m > 0) | (pos > 0))
                def _():
                    pltpu.make_async_copy(
                        gst_hbm.at[pl.ds(0, _CH)],
                        rows.at[pl.ds(roff_n, _CH)], ssems[nslot]).wait()

                @pl.when(notlast2)
                def _():
                    pltpu.async_copy(gst_hbm.at[srcv.at[pl.ds(sb2, _CH)]],
                                     rows.at[pl.ds(roff_n, _CH)],
                                     gsems[nslot])

                pltpu.make_async_copy(gst_hbm.at[srcv.at[pl.ds(sb, _CH)]],
                                      rows.at[pl.ds(roff, _CH)],
                                      gsems[slot]).wait()
                for q in range(2):
                    idx16 = dstv[pl.ds(sb + q * 16, 16)]
                    pltpu.async_copy(rows.at[pl.ds(roff + q * 16, 16)],
                                     acc.at[idx16], ssems[slot], add=True)

            par = lax.rem(m + pos, 3)
            for slot in range(3):
                @pl.when(par == slot)
                def _(slot=slot):
                    work(slot, (slot + 2) % 3)

            return 0

        lax.fori_loop(0, _CPB, chunk, 0)
        return 0

    lax.fori_loop(0, _NSLAB, slab, 0)
    pltpu.make_async_copy(gst_hbm.at[pl.ds(0, _CH)],
                          rows.at[pl.ds(0, _CH)], ssem0).wait()
    plsc.subcore_barrier()

    def ochunk(k, _):
        cid = s + _NS * k

        @pl.when(cid < _NOCH)
        def _():
            sl = pl.ds(cid * _OCH, _OCH)

            @pl.when(c == 0)
            def _():
                pltpu.sync_copy(acc.at[sl], out0.at[sl])

            @pl.when(c == 1)
            def _():
                pltpu.sync_copy(acc.at[sl], out1.at[sl])

        return 0

    lax.fori_loop(0, _NOCH // _NS + 1, ochunk, 0)


_GRID = 10
_RBLK = _N // _GRID
_IBLK = _E // _D // _GRID


def _dot(a, b, dims):
    return lax.dot_general(a, b, (dims, ((), ())),
                           precision=lax.Precision.HIGHEST,
                           preferred_element_type=_f32)


def _half_map(i):
    return (lax.rem(i, _GRID // 2), 0)


def _tc_prep_body(dega_ref, degb_ref, feats_ref, dis_ref, gst_ref):
    deg = dega_ref[...] + degb_ref[...]
    safe = jnp.where(deg > 0, deg, 1.0)
    dis = jnp.where(deg > 0, lax.rsqrt(safe), 0.0)
    dis_ref[...] = dis
    g = feats_ref[...] * dis
    gst_ref[0] = g
    gst_ref[1] = -g


_tc_prep = pl.pallas_call(
    _tc_prep_body,
    grid=(_GRID,),
    in_specs=[
        pl.BlockSpec((_RBLK, 1), lambda i: (i, 0)),
        pl.BlockSpec((_RBLK, 1), lambda i: (i, 0)),
        pl.BlockSpec((_RBLK, _D), lambda i: (i, 0)),
    ],
    out_specs=[
        pl.BlockSpec((_RBLK, 1), lambda i: (i, 0)),
        pl.BlockSpec((2, _RBLK, _D), lambda i: (0, i, 0)),
    ],
    out_shape=[
        jax.ShapeDtypeStruct((_N, 1), _f32),
        jax.ShapeDtypeStruct((2, _N, _D), _f32),
    ],
)


def _tc_idx_body(src_ref, dst_ref, gi1_ref, df_ref):
    srci = src_ref[...]
    dsti = dst_ref[...]
    hi = dsti >= _HALF
    gi1_ref[...] = jnp.where(hi, srci + _N, srci)
    df_ref[...] = jnp.where(hi, dsti - _HALF, dsti)


_tc_idx = pl.pallas_call(
    _tc_idx_body,
    out_shape=[
        jax.ShapeDtypeStruct((_E // _D, _D), jnp.int32),
        jax.ShapeDtypeStruct((_E // _D, _D), jnp.int32),
    ],
)


def _fold_decode(a0, a1, i):
    sign = jnp.where(i < _GRID // 2, 0.5, -0.5)
    return 0.5 * a0 + sign * a1


def _tc_mid_a_body(a0_ref, a1_ref, dis_ref, gst1_ref):
    i = pl.program_id(0)
    s0 = _fold_decode(a0_ref[...], a1_ref[...], i)
    dis = dis_ref[...]
    g1 = -dis * dis * s0
    gst1_ref[0] = g1
    gst1_ref[1] = -g1


_tc_mid_a = pl.pallas_call(
    _tc_mid_a_body,
    grid=(_GRID,),
    in_specs=[
        pl.BlockSpec((_RBLK, _D), _half_map),
        pl.BlockSpec((_RBLK, _D), _half_map),
        pl.BlockSpec((_RBLK, 1), lambda i: (i, 0)),
    ],
    out_specs=pl.BlockSpec((2, _RBLK, _D), lambda i: (0, i, 0)),
    out_shape=jax.ShapeDtypeStruct((2, _N, _D), _f32),
)


def _tc_mid_b_body(a0_ref, a1_ref, dis_ref, feats_ref, w0_ref, w1_ref,
                   out01_ref):
    i = pl.program_id(0)
    s0 = _fold_decode(a0_ref[...], a1_ref[...], i)
    tx1 = -dis_ref[...] * s0
    out01_ref[...] = (_dot(feats_ref[...], w0_ref[...], ((1,), (0,)))
                      + _dot(tx1, w1_ref[...], ((1,), (0,))))


_tc_mid_b = pl.pallas_call(
    _tc_mid_b_body,
    grid=(_GRID,),
    in_specs=[
        pl.BlockSpec((_RBLK, _D), _half_map),
        pl.BlockSpec((_RBLK, _D), _half_map),
        pl.BlockSpec((_RBLK, 1), lambda i: (i, 0)),
        pl.BlockSpec((_RBLK, _D), lambda i: (i, 0)),
        pl.BlockSpec((_D, _D), lambda i: (0, 0)),
        pl.BlockSpec((_D, _D), lambda i: (0, 0)),
    ],
    out_specs=pl.BlockSpec((_RBLK, _D), lambda i: (i, 0)),
    out_shape=jax.ShapeDtypeStruct((_N, _D), _f32),
)


def _tc_final_body(a0_ref, a1_ref, dis_ref, feats_ref, out01_ref, batch_ref,
                   w2_ref, bch_ref, gnw_ref, gnb_ref, gnms_ref, ow_ref, ob_ref,
                   out_ref, sums1, sums2, cnt, maxz):
    i = pl.program_id(0)

    @pl.when(i == 0)
    def _():
        sums1[...] = jnp.zeros_like(sums1)
        sums2[...] = jnp.zeros_like(sums2)
        cnt[...] = jnp.zeros_like(cnt)
        maxz[...] = jnp.full_like(maxz, -3.0e38)

    s1 = _fold_decode(a0_ref[...], a1_ref[...], i)
    tx2 = -2.0 * dis_ref[...] * s1 - feats_ref[...]
    x1 = out01_ref[...] + _dot(tx2, w2_ref[...], ((1,), (0,))) + bch_ref[...]
    x1 = jnp.clip(x1, -1.0, 1.0)

    bcol = batch_ref[...]
    iota = lax.broadcasted_iota(jnp.int32, (1, _B), 1).astype(_f32)
    onehot = (bcol == iota).astype(_f32)
    cnt[...] += _dot(onehot, jnp.ones((_RBLK, 1), _f32), ((0,), (0,)))
    sums1[...] += _dot(onehot, x1, ((0,), (0,)))
    sums2[...] += _dot(onehot, x1 * x1, ((0,), (0,)))

    sign = jnp.where(gnw_ref[...] >= 0, 1.0, -1.0)
    z = x1 * sign
    for b in range(_B):
        mb = jnp.max(jnp.where(bcol == float(b), z, -3.0e38), axis=0)
        maxz[b, :] = jnp.maximum(maxz[b, :], mb)

    @pl.when(i == _GRID - 1)
    def _():
        cn = cnt[...]
        m1 = sums1[...] / cn
        m2 = sums2[...] / cn
        ms = gnms_ref[...]
        w = gnw_ref[...]
        var = m2 - (2.0 * ms - ms * ms) * m1 * m1
        std = jnp.sqrt(var + _EPS)
        pooled = (jnp.abs(w) * maxz[...] - w * ms * m1) / std + gnb_ref[...]
        out_ref[...] = _dot(pooled, ow_ref[...], ((1,), (1,))) + ob_ref[...]


_tc_final = pl.pallas_call(
    _tc_final_body,
    grid=(_GRID,),
    in_specs=[
        pl.BlockSpec((_RBLK, _D), _half_map),
        pl.BlockSpec((_RBLK, _D), _half_map),
        pl.BlockSpec((_RBLK, 1), lambda i: (i, 0)),
        pl.BlockSpec((_RBLK, _D), lambda i: (i, 0)),
        pl.BlockSpec((_RBLK, _D), lambda i: (i, 0)),
        pl.BlockSpec((_RBLK, 1), lambda i: (i, 0)),
        pl.BlockSpec((_D, _D), lambda i: (0, 0)),
        pl.BlockSpec((1, _D), lambda i: (0, 0)),
        pl.BlockSpec((1, _D), lambda i: (0, 0)),
        pl.BlockSpec((1, _D), lambda i: (0, 0)),
        pl.BlockSpec((1, _D), lambda i: (0, 0)),
        pl.BlockSpec((_D, _D), lambda i: (0, 0)),
        pl.BlockSpec((1, _D), lambda i: (0, 0)),
    ],
    out_specs=pl.BlockSpec((_B, _D), lambda i: (0, 0)),
    out_shape=jax.ShapeDtypeStruct((_B, _D), _f32),
    scratch_shapes=[
        pltpu.VMEM((_B, _D), _f32),
        pltpu.VMEM((_B, _D), _f32),
        pltpu.VMEM((_B, 1), _f32),
        pltpu.VMEM((_B, _D), _f32),
    ],
)


def kernel(x, edge_index, W_cheb, b_cheb, gn_weight, gn_bias, gn_mean_scale,
           out_W, out_b):
    feats = x[:, :_D]
    batchf = x[:, -1:]
    src = edge_index[0]
    dst = edge_index[1]
    srcd = src.reshape(_NW, _DNCH, _DCH)
    src2 = src.reshape(_E // _D, _D)
    dst2 = dst.reshape(_E // _D, _D)
    gi0 = src

    dega, degb = _sc_deg(srcd)
    gi1, df = _tc_idx(src2, dst2)
    dis, gst0 = _tc_prep(dega.reshape(_N, 1), degb.reshape(_N, 1), feats)
    gi1 = gi1.reshape(_E)
    df = df.reshape(_E)

    a0, a1 = _sc_scatter(gst0.reshape(2 * _N, _D), gi0, gi1, df)
    gst1 = _tc_mid_a(a0, a1, dis)
    b0, b1 = _sc_scatter(gst1.reshape(2 * _N, _D), gi0, gi1, df)
    out01 = _tc_mid_b(a0, a1, dis, feats, W_cheb[0], W_cheb[1])
    row = lambda v: v.reshape(1, _D)
    return _tc_final(b0, b1, dis, feats, out01, batchf, W_cheb[2],
                     row(b_cheb), row(gn_weight), row(gn_bias),
                     row(gn_mean_scale), out_W, row(out_b))
